# Initial kernel scaffold; baseline (speedup 1.0000x reference)
import jax
import jax.numpy as jnp
from jax.experimental import pallas as pl


def kernel(query_points, codes, W_embed, b_embed, Wm, bm, We, Wu, bu, Wn, W_out, b_out):
    raise NotImplementedError("write your pallas kernel here")


# speedup vs baseline: 3.4279x; 3.4279x over previous
import functools

import jax
import jax.numpy as jnp
from jax import lax
from jax.experimental import pallas as pl
from jax.experimental.pallas import tpu as pltpu
from jax.experimental.pallas import tpu_sc as plsc

_B = 2
_N = 4096
_G = 16
_NG = _G ** 3
_CODE = 128
_K = 32
_H = 240
_L = 3
_SH = 9
_NA = 4
_CUTOFF = 8.0
_SPACING = 1.5
_ORIGIN = -(_G - 1) / 2.0 * _SPACING
_BN = _B * _N
_BNG = _B * _NG
_E = _BN * _K

_HP = 256
_QB = 128
_CAND = 256
_SB = 6
_RB = 512
_QB2 = 128
_EB = _QB2 * _K

_NC = 2
_NS = 16
_NW = _NC * _NS
_CH = 128

_F32 = jnp.float32
_HIGH = lax.Precision.HIGHEST


def _silu(x):
    return x * jax.nn.sigmoid(x)


def _knn_body(q_ref, idx_ref, ea_ref, mask_ref, na_ref):
    qx = q_ref[:, 0:1]
    qy = q_ref[:, 1:2]
    qz = q_ref[:, 2:3]

    j = lax.broadcasted_iota(jnp.int32, (_QB, _CAND), 1)
    jf = j.astype(_F32)
    ox = jnp.floor((jf + 0.5) * (1.0 / 36.0))
    jm = jf - 36.0 * ox
    oy = jnp.floor((jm + 0.5) * (1.0 / 6.0))
    oz = jm - 6.0 * oy

    inv_sp = 1.0 / _SPACING
    bx = jnp.clip(jnp.floor((qx - _ORIGIN) * inv_sp).astype(jnp.int32) - 2, 0, _G - _SB)
    by = jnp.clip(jnp.floor((qy - _ORIGIN) * inv_sp).astype(jnp.int32) - 2, 0, _G - _SB)
    bz = jnp.clip(jnp.floor((qz - _ORIGIN) * inv_sp).astype(jnp.int32) - 2, 0, _G - _SB)

    cx = bx.astype(_F32) + ox
    cy = by.astype(_F32) + oy
    cz = bz.astype(_F32) + oz
    rx = (cx * _SPACING + _ORIGIN) - qx
    ry = (cy * _SPACING + _ORIGIN) - qy
    rz = (cz * _SPACING + _ORIGIN) - qz
    d2 = rx * rx + ry * ry + rz * rz

    valid = j < (_SB * _SB * _SB)
    gidx = (cx * 256.0 + cy * 16.0 + cz).astype(jnp.int32)
    gidx = jnp.where(valid, gidx, 100000 + j)
    big = jnp.float32(3.0e38)
    work = jnp.where(valid, d2, big)

    kiota = lax.broadcasted_iota(jnp.int32, (_QB, _K), 1)
    selg = jnp.zeros((_QB, _K), jnp.int32)
    srx = jnp.zeros((_QB, _K), _F32)
    sry = jnp.zeros((_QB, _K), _F32)
    srz = jnp.zeros((_QB, _K), _F32)
    sd2 = jnp.zeros((_QB, _K), _F32)
    imax = jnp.int32(2 ** 31 - 1)
    for k in range(_K):
        m = jnp.min(work, axis=1, keepdims=True)
        sg = jnp.min(jnp.where(work == m, gidx, imax), axis=1, keepdims=True)
        onehot = gidx == sg
        rx_k = jnp.sum(jnp.where(onehot, rx, 0.0), axis=1, keepdims=True)
        ry_k = jnp.sum(jnp.where(onehot, ry, 0.0), axis=1, keepdims=True)
        rz_k = jnp.sum(jnp.where(onehot, rz, 0.0), axis=1, keepdims=True)
        work = jnp.where(onehot, big, work)
        col = kiota == k
        selg = jnp.where(col, sg, selg)
        srx = jnp.where(col, rx_k, srx)
        sry = jnp.where(col, ry_k, sry)
        srz = jnp.where(col, rz_k, srz)
        sd2 = jnp.where(col, m, sd2)

    dist = jnp.sqrt(sd2 + 1e-12)
    maskv = (dist <= _CUTOFF).astype(_F32)
    x = srx / dist
    y = sry / dist
    z = srz / dist
    c0 = 0.28209479177387814
    c1 = 0.4886025119029199
    c2 = 1.0925484305920792
    c20 = 0.31539156525252005
    c22 = 0.5462742152960396
    ea = jnp.stack([
        jnp.full((_QB, _K), c0, _F32),
        c1 * y, c1 * z, c1 * x,
        c2 * x * y, c2 * y * z, c20 * (3.0 * z * z - 1.0), c2 * x * z,
        c22 * (x * x - y * y)
    ], axis=0)

    boff = (pl.program_id(0) * _QB // _N) * _NG
    idx_ref[...] = selg + boff
    ea_ref[...] = ea
    mask_ref[...] = maskv
    den = jnp.maximum(jnp.sum(maskv, axis=1, keepdims=True), 1.0)
    na_cols = [jnp.ones((_QB, 1), _F32)]
    for i in range(1, _SH):
        na_cols.append(jnp.sum(ea[i] * maskv, axis=1, keepdims=True) / den)
    na_ref[...] = jnp.concatenate(na_cols, axis=1)


def _knn(qflat):
    grid = (_BN // _QB,)
    return pl.pallas_call(
        _knn_body,
        grid=grid,
        in_specs=[pl.BlockSpec((_QB, 3), lambda i: (i, 0))],
        out_specs=[
            pl.BlockSpec((_QB, _K), lambda i: (i, 0)),
            pl.BlockSpec((_SH, _QB, _K), lambda i: (0, i, 0)),
            pl.BlockSpec((_QB, _K), lambda i: (i, 0)),
            pl.BlockSpec((_QB, _SH), lambda i: (i, 0)),
        ],
        out_shape=[
            jax.ShapeDtypeStruct((_BN, _K), jnp.int32),
            jax.ShapeDtypeStruct((_SH, _BN, _K), _F32),
            jax.ShapeDtypeStruct((_BN, _K), _F32),
            jax.ShapeDtypeStruct((_BN, _SH), _F32),
        ],
    )(qflat)


def _embed_body(codes_ref, w_ref, b_ref, hg_ref, hq_ref):
    z = jnp.dot(codes_ref[...], w_ref[...], precision=_HIGH,
                preferred_element_type=_F32) + b_ref[...]
    hg_ref[...] = _silu(z)
    hq_ref[...] = jnp.broadcast_to(_silu(b_ref[...]), (_RB, _H))


def _embed(codes_flat, W_embed, b_embed):
    grid = (_BNG // _RB,)
    return pl.pallas_call(
        _embed_body,
        grid=grid,
        in_specs=[
            pl.BlockSpec((_RB, _CODE), lambda i: (i, 0)),
            pl.BlockSpec((_CODE, _H), lambda i: (0, 0)),
            pl.BlockSpec((1, _H), lambda i: (0, 0)),
        ],
        out_specs=[
            pl.BlockSpec((_RB, _H), lambda i: (i, 0)),
            pl.BlockSpec((_RB, _H), lambda i: (i, 0)),
        ],
        out_shape=[
            jax.ShapeDtypeStruct((_BNG, _H), _F32),
            jax.ShapeDtypeStruct((_BN, _H), _F32),
        ],
    )(codes_flat, W_embed, b_embed)


def _pre_body(hg_ref, hq_ref, wt_ref, wb_ref, bm_ref, gp_ref, cq_ref):
    gp = jnp.dot(hg_ref[...], wt_ref[...], precision=_HIGH,
                 preferred_element_type=_F32)
    gp_ref[...] = jnp.concatenate(
        [gp, jnp.zeros((_RB, _HP - _H), _F32)], axis=1)
    cq_ref[...] = jnp.dot(hq_ref[...], wb_ref[...], precision=_HIGH,
                          preferred_element_type=_F32) + bm_ref[...]


def _pre(h_g, h_q, wt, wb, bm):
    grid = (_BNG // _RB,)
    return pl.pallas_call(
        _pre_body,
        grid=grid,
        in_specs=[
            pl.BlockSpec((_RB, _H), lambda i: (i, 0)),
            pl.BlockSpec((_RB, _H), lambda i: (i, 0)),
            pl.BlockSpec((_H, _H), lambda i: (0, 0)),
            pl.BlockSpec((_H, _H), lambda i: (0, 0)),
            pl.BlockSpec((1, _H), lambda i: (0, 0)),
        ],
        out_specs=[
            pl.BlockSpec((_RB, _HP), lambda i: (i, 0)),
            pl.BlockSpec((_RB, _H), lambda i: (i, 0)),
        ],
        out_shape=[
            jax.ShapeDtypeStruct((_BNG, _HP), _F32),
            jax.ShapeDtypeStruct((_BN, _H), _F32),
        ],
    )(h_g, h_q, wt, wb, bm)


def _sc_gather_body(table_ref, idx_ref, out_ref, idx_v, rows_v, sem):
    wid = lax.axis_index("s") * _NC + lax.axis_index("c")
    per_w = _E // _NW
    base = wid * per_w

    def chunk(i, carry):
        off = base + i * _CH
        pltpu.sync_copy(idx_ref.at[pl.ds(off, _CH)], idx_v)
        pltpu.async_copy(table_ref.at[idx_v], rows_v, sem).wait()
        pltpu.sync_copy(rows_v, out_ref.at[pl.ds(off, _CH)])
        return carry

    lax.fori_loop(0, per_w // _CH, chunk, 0)


@functools.lru_cache(maxsize=1)
def _sc_gather_fn():
    return pl.kernel(
        _sc_gather_body,
        out_type=jax.ShapeDtypeStruct((_E, _HP), _F32),
        mesh=plsc.VectorSubcoreMesh(core_axis_name="c", subcore_axis_name="s"),
        scratch_types=[
            pltpu.VMEM((_CH,), jnp.int32),
            pltpu.VMEM((_CH, _HP), _F32),
            pltpu.SemaphoreType.DMA,
        ],
    )


def _sc_gather(table, idx):
    return _sc_gather_fn()(table, idx)


def _edge_body(g_ref, ea_ref, cq_ref, mask_ref, we_ref, agg_ref):
    g3 = g_ref[:, :_H].reshape(_QB2, _K, _H)
    pre = g3 + cq_ref[...].reshape(_QB2, 1, _H)
    m = _silu(pre)
    gate = jax.nn.sigmoid(
        jnp.dot(ea_ref[...], we_ref[...], precision=_HIGH,
                preferred_element_type=_F32)) * mask_ref[...]
    m = m * gate.reshape(_QB2, _K, _H)
    agg_ref[...] = jnp.sum(m, axis=1)


def _edge(gathered, ea_flat, c_q, mask_flat, we):
    grid = (_BN // _QB2,)
    return pl.pallas_call(
        _edge_body,
        grid=grid,
        in_specs=[
            pl.BlockSpec((_EB, _HP), lambda i: (i, 0)),
            pl.BlockSpec((_EB, _SH), lambda i: (i, 0)),
            pl.BlockSpec((_QB2, _H), lambda i: (i, 0)),
            pl.BlockSpec((_EB, 1), lambda i: (i, 0)),
            pl.BlockSpec((_SH, _H), lambda i: (0, 0)),
        ],
        out_specs=[pl.BlockSpec((_QB2, _H), lambda i: (i, 0))],
        out_shape=[jax.ShapeDtypeStruct((_BN, _H), _F32)],
    )(gathered, ea_flat, c_q, mask_flat, we)[0]


def _update_body(hq_ref, agg_ref, na_ref, hg_ref, wt_ref, wb_ref, bu_ref,
                 wn_ref, hqo_ref, hgo_ref):
    hq = hq_ref[...]
    uq = _silu(jnp.dot(hq, wt_ref[...], precision=_HIGH,
                       preferred_element_type=_F32)
               + jnp.dot(agg_ref[...], wb_ref[...], precision=_HIGH,
                         preferred_element_type=_F32) + bu_ref[...])
    gq = jax.nn.sigmoid(jnp.dot(na_ref[...], wn_ref[...], precision=_HIGH,
                                preferred_element_type=_F32))
    hq2 = hq + uq * gq
    hqo_ref[...] = hq2 / jnp.sqrt(
        jnp.mean(hq2 * hq2, axis=-1, keepdims=True) + 1e-6)

    hg = hg_ref[...]
    ug = _silu(jnp.dot(hg, wt_ref[...], precision=_HIGH,
                       preferred_element_type=_F32) + bu_ref[...])
    gg = jax.nn.sigmoid(wn_ref[0:1, :])
    hg2 = hg + ug * gg
    hgo_ref[...] = hg2 / jnp.sqrt(
        jnp.mean(hg2 * hg2, axis=-1, keepdims=True) + 1e-6)


def _update(h_q, agg, na, h_g, wt, wb, bu, wn):
    grid = (_BNG // _RB,)
    return pl.pallas_call(
        _update_body,
        grid=grid,
        in_specs=[
            pl.BlockSpec((_RB, _H), lambda i: (i, 0)),
            pl.BlockSpec((_RB, _H), lambda i: (i, 0)),
            pl.BlockSpec((_RB, _SH), lambda i: (i, 0)),
            pl.BlockSpec((_RB, _H), lambda i: (i, 0)),
            pl.BlockSpec((_H, _H), lambda i: (0, 0)),
            pl.BlockSpec((_H, _H), lambda i: (0, 0)),
            pl.BlockSpec((1, _H), lambda i: (0, 0)),
            pl.BlockSpec((_SH, _H), lambda i: (0, 0)),
        ],
        out_specs=[
            pl.BlockSpec((_RB, _H), lambda i: (i, 0)),
            pl.BlockSpec((_RB, _H), lambda i: (i, 0)),
        ],
        out_shape=[
            jax.ShapeDtypeStruct((_BN, _H), _F32),
            jax.ShapeDtypeStruct((_BNG, _H), _F32),
        ],
    )(h_q, agg, na, h_g, wt, wb, bu, wn)


def _out_body(hq_ref, w_ref, b_ref, o_ref):
    o_ref[...] = jnp.dot(hq_ref[...], w_ref[...], precision=_HIGH,
                         preferred_element_type=_F32) + b_ref[...]


def _final(h_q, W_out, b_out):
    grid = (_BN // _RB,)
    return pl.pallas_call(
        _out_body,
        grid=grid,
        in_specs=[
            pl.BlockSpec((_RB, _H), lambda i: (i, 0)),
            pl.BlockSpec((_H, _NA * 3), lambda i: (0, 0)),
            pl.BlockSpec((1, _NA * 3), lambda i: (0, 0)),
        ],
        out_specs=[pl.BlockSpec((_RB, _NA * 3), lambda i: (i, 0))],
        out_shape=[jax.ShapeDtypeStruct((_BN, _NA * 3), _F32)],
    )(h_q, W_out, b_out)[0]


def kernel(query_points, codes, W_embed, b_embed, Wm, bm, We, Wu, bu, Wn,
           W_out, b_out):
    qflat = query_points.reshape(_BN, 3)
    codes_flat = codes.reshape(_BNG, _CODE)

    idxg, ea_t, maskm, na = _knn(qflat)
    ea_flat = ea_t.transpose(1, 2, 0).reshape(_E, _SH)
    mask_flat = maskm.reshape(_E, 1)
    idx_flat = idxg.reshape(_E)

    h_g, h_q = _embed(codes_flat, W_embed, b_embed.reshape(1, _H))

    for l in range(_L):
        g_proj, c_q = _pre(h_g, h_q, Wm[l, :_H], Wm[l, _H:],
                           bm[l].reshape(1, _H))
        gathered = _sc_gather(g_proj, idx_flat)
        agg = _edge(gathered, ea_flat, c_q, mask_flat, We[l])
        h_q, h_g = _update(h_q, agg, na, h_g, Wu[l, :_H], Wu[l, _H:],
                           bu[l].reshape(1, _H), Wn[l])

    out = _final(h_q, W_out, b_out.reshape(1, _NA * 3))
    return out.reshape(_B, _N, _NA, 3)


# speedup vs baseline: 5.5325x; 1.6140x over previous
import functools

import jax
import jax.numpy as jnp
from jax import lax
from jax.experimental import pallas as pl
from jax.experimental.pallas import tpu as pltpu
from jax.experimental.pallas import tpu_sc as plsc

_B = 2
_N = 4096
_G = 16
_NG = _G ** 3
_CODE = 128
_K = 32
_H = 240
_L = 3
_SH = 9
_NA = 4
_CUTOFF = 8.0
_SPACING = 1.5
_ORIGIN = -(_G - 1) / 2.0 * _SPACING
_BN = _B * _N
_BNG = _B * _NG
_E = _BN * _K

_HP = 256
_QB = 128
_CAND = 256
_SB = 6
_RB = 512
_QB2 = 128
_EB = _QB2 * _K

_NC = 2
_NS = 16
_NW = _NC * _NS
_CH = 128

_F32 = jnp.float32
_HIGH = lax.Precision.HIGHEST


def _silu(x):
    return x * jax.nn.sigmoid(x)


def _rmsnorm(h):
    return h / jnp.sqrt(jnp.mean(h * h, axis=-1, keepdims=True) + 1e-6)


def _knn_body(q_ref, idx_ref, ea_ref, mask_ref, na_ref):
    qx = q_ref[:, 0:1]
    qy = q_ref[:, 1:2]
    qz = q_ref[:, 2:3]

    j = lax.broadcasted_iota(jnp.int32, (_QB, _CAND), 1)
    jf = j.astype(_F32)
    ox = jnp.floor((jf + 0.5) * (1.0 / 36.0))
    jm = jf - 36.0 * ox
    oy = jnp.floor((jm + 0.5) * (1.0 / 6.0))
    oz = jm - 6.0 * oy

    inv_sp = 1.0 / _SPACING
    bx = jnp.clip(jnp.floor((qx - _ORIGIN) * inv_sp).astype(jnp.int32) - 2, 0, _G - _SB)
    by = jnp.clip(jnp.floor((qy - _ORIGIN) * inv_sp).astype(jnp.int32) - 2, 0, _G - _SB)
    bz = jnp.clip(jnp.floor((qz - _ORIGIN) * inv_sp).astype(jnp.int32) - 2, 0, _G - _SB)

    cx = bx.astype(_F32) + ox
    cy = by.astype(_F32) + oy
    cz = bz.astype(_F32) + oz
    rx = (cx * _SPACING + _ORIGIN) - qx
    ry = (cy * _SPACING + _ORIGIN) - qy
    rz = (cz * _SPACING + _ORIGIN) - qz
    d2 = rx * rx + ry * ry + rz * rz

    valid = j < (_SB * _SB * _SB)
    gidx = (cx * 256.0 + cy * 16.0 + cz).astype(jnp.int32)
    gidx = jnp.where(valid, gidx, 100000 + j)
    big = jnp.float32(3.0e38)
    work = jnp.where(valid, d2, big)

    kiota = lax.broadcasted_iota(jnp.int32, (_QB, _K), 1)
    selg = jnp.zeros((_QB, _K), jnp.int32)
    srx = jnp.zeros((_QB, _K), _F32)
    sry = jnp.zeros((_QB, _K), _F32)
    srz = jnp.zeros((_QB, _K), _F32)
    sd2 = jnp.zeros((_QB, _K), _F32)
    imax = jnp.int32(2 ** 31 - 1)
    for k in range(_K):
        m = jnp.min(work, axis=1, keepdims=True)
        sg = jnp.min(jnp.where(work == m, gidx, imax), axis=1, keepdims=True)
        onehot = gidx == sg
        rx_k = jnp.sum(jnp.where(onehot, rx, 0.0), axis=1, keepdims=True)
        ry_k = jnp.sum(jnp.where(onehot, ry, 0.0), axis=1, keepdims=True)
        rz_k = jnp.sum(jnp.where(onehot, rz, 0.0), axis=1, keepdims=True)
        work = jnp.where(onehot, big, work)
        col = kiota == k
        selg = jnp.where(col, sg, selg)
        srx = jnp.where(col, rx_k, srx)
        sry = jnp.where(col, ry_k, sry)
        srz = jnp.where(col, rz_k, srz)
        sd2 = jnp.where(col, m, sd2)

    dist = jnp.sqrt(sd2 + 1e-12)
    maskv = (dist <= _CUTOFF).astype(_F32)
    x = srx / dist
    y = sry / dist
    z = srz / dist
    c0 = 0.28209479177387814
    c1 = 0.4886025119029199
    c2 = 1.0925484305920792
    c20 = 0.31539156525252005
    c22 = 0.5462742152960396
    comps = [
        jnp.full((_QB, _K), c0, _F32),
        c1 * y, c1 * z, c1 * x,
        c2 * x * y, c2 * y * z, c20 * (3.0 * z * z - 1.0), c2 * x * z,
        c22 * (x * x - y * y)
    ]

    boff = (pl.program_id(0) * _QB // _N) * _NG
    idx_ref[...] = selg + boff
    ea_ref[...] = jnp.stack(comps, axis=-1).reshape(_QB * _K, _SH)
    mask_ref[...] = maskv
    den = jnp.maximum(jnp.sum(maskv, axis=1, keepdims=True), 1.0)
    na_cols = [jnp.ones((_QB, 1), _F32)]
    for i in range(1, _SH):
        na_cols.append(jnp.sum(comps[i] * maskv, axis=1, keepdims=True) / den)
    na_ref[...] = jnp.concatenate(na_cols, axis=1)


def _knn(qflat):
    grid = (_BN // _QB,)
    return pl.pallas_call(
        _knn_body,
        grid=grid,
        in_specs=[pl.BlockSpec((_QB, 3), lambda i: (i, 0))],
        out_specs=[
            pl.BlockSpec((_QB, _K), lambda i: (i, 0)),
            pl.BlockSpec((_QB * _K, _SH), lambda i: (i, 0)),
            pl.BlockSpec((_QB, _K), lambda i: (i, 0)),
            pl.BlockSpec((_QB, _SH), lambda i: (i, 0)),
        ],
        out_shape=[
            jax.ShapeDtypeStruct((_BN, _K), jnp.int32),
            jax.ShapeDtypeStruct((_E, _SH), _F32),
            jax.ShapeDtypeStruct((_BN, _K), _F32),
            jax.ShapeDtypeStruct((_BN, _SH), _F32),
        ],
    )(qflat)


def _embed_body(codes_ref, w_ref, b_ref, wmb_ref, bm_ref, hg_ref, hq_ref,
                cq_ref):
    z = jnp.dot(codes_ref[...], w_ref[...], precision=_HIGH,
                preferred_element_type=_F32) + b_ref[...]
    hg_ref[...] = _silu(z)
    hq_row = _silu(b_ref[...])
    hq_ref[...] = jnp.broadcast_to(hq_row, (_RB, _H))
    cq_row = jnp.dot(hq_row, wmb_ref[...], precision=_HIGH,
                     preferred_element_type=_F32) + bm_ref[...]
    cq_ref[...] = jnp.broadcast_to(cq_row, (_RB, _H))


def _embed(codes_flat, W_embed, b_embed, wmb0, bm0):
    grid = (_BNG // _RB,)
    return pl.pallas_call(
        _embed_body,
        grid=grid,
        in_specs=[
            pl.BlockSpec((_RB, _CODE), lambda i: (i, 0)),
            pl.BlockSpec((_CODE, _H), lambda i: (0, 0)),
            pl.BlockSpec((1, _H), lambda i: (0, 0)),
            pl.BlockSpec((_H, _H), lambda i: (0, 0)),
            pl.BlockSpec((1, _H), lambda i: (0, 0)),
        ],
        out_specs=[
            pl.BlockSpec((_RB, _H), lambda i: (i, 0)),
            pl.BlockSpec((_RB, _H), lambda i: (i, 0)),
            pl.BlockSpec((_RB, _H), lambda i: (i, 0)),
        ],
        out_shape=[
            jax.ShapeDtypeStruct((_BNG, _H), _F32),
            jax.ShapeDtypeStruct((_BN, _H), _F32),
            jax.ShapeDtypeStruct((_BN, _H), _F32),
        ],
    )(codes_flat, W_embed, b_embed, wmb0, bm0)


def _grid_body(hg_ref, wmt_ref, wut_ref, bu_ref, wn_ref, gp_ref, hgo_ref):
    hg = hg_ref[...]
    gp = jnp.dot(hg, wmt_ref[...], precision=_HIGH,
                 preferred_element_type=_F32)
    gp_ref[...] = jnp.concatenate(
        [gp, jnp.zeros((_RB, _HP - _H), _F32)], axis=1)
    ug = _silu(jnp.dot(hg, wut_ref[...], precision=_HIGH,
                       preferred_element_type=_F32) + bu_ref[...])
    gg = jax.nn.sigmoid(wn_ref[0:1, :])
    hgo_ref[...] = _rmsnorm(hg + ug * gg)


def _grid_step(h_g, wmt, wut, bu, wn):
    grid = (_BNG // _RB,)
    return pl.pallas_call(
        _grid_body,
        grid=grid,
        in_specs=[
            pl.BlockSpec((_RB, _H), lambda i: (i, 0)),
            pl.BlockSpec((_H, _H), lambda i: (0, 0)),
            pl.BlockSpec((_H, _H), lambda i: (0, 0)),
            pl.BlockSpec((1, _H), lambda i: (0, 0)),
            pl.BlockSpec((_SH, _H), lambda i: (0, 0)),
        ],
        out_specs=[
            pl.BlockSpec((_RB, _HP), lambda i: (i, 0)),
            pl.BlockSpec((_RB, _H), lambda i: (i, 0)),
        ],
        out_shape=[
            jax.ShapeDtypeStruct((_BNG, _HP), _F32),
            jax.ShapeDtypeStruct((_BNG, _H), _F32),
        ],
    )(h_g, wmt, wut, bu, wn)


def _sc_gather_body(table_ref, idx_ref, out_ref, idx_v, rows_v, sem):
    wid = lax.axis_index("s") * _NC + lax.axis_index("c")
    per_w = _E // _NW
    base = wid * per_w

    def chunk(i, carry):
        off = base + i * _CH
        pltpu.sync_copy(idx_ref.at[pl.ds(off, _CH)], idx_v)
        pltpu.async_copy(table_ref.at[idx_v], rows_v, sem).wait()
        pltpu.sync_copy(rows_v, out_ref.at[pl.ds(off, _CH)])
        return carry

    lax.fori_loop(0, per_w // _CH, chunk, 0)


@functools.lru_cache(maxsize=1)
def _sc_gather_fn():
    return pl.kernel(
        _sc_gather_body,
        out_type=jax.ShapeDtypeStruct((_E, _HP), _F32),
        mesh=plsc.VectorSubcoreMesh(core_axis_name="c", subcore_axis_name="s"),
        scratch_types=[
            pltpu.VMEM((_CH,), jnp.int32),
            pltpu.VMEM((_CH, _HP), _F32),
            pltpu.SemaphoreType.DMA,
        ],
    )


def _sc_gather(table, idx):
    return _sc_gather_fn()(table, idx)


def _edge_body(g_ref, ea_ref, cq_ref, mask_ref, we_ref, agg_ref):
    g3 = g_ref[:, :_H].reshape(_QB2, _K, _H)
    pre = g3 + cq_ref[...].reshape(_QB2, 1, _H)
    m = _silu(pre)
    gate = jax.nn.sigmoid(
        jnp.dot(ea_ref[...], we_ref[...], precision=_HIGH,
                preferred_element_type=_F32)) * mask_ref[...]
    m = m * gate.reshape(_QB2, _K, _H)
    agg_ref[...] = jnp.sum(m, axis=1)


def _edge(gathered, ea_flat, c_q, mask_flat, we):
    grid = (_BN // _QB2,)
    return pl.pallas_call(
        _edge_body,
        grid=grid,
        in_specs=[
            pl.BlockSpec((_EB, _HP), lambda i: (i, 0)),
            pl.BlockSpec((_EB, _SH), lambda i: (i, 0)),
            pl.BlockSpec((_QB2, _H), lambda i: (i, 0)),
            pl.BlockSpec((_EB, 1), lambda i: (i, 0)),
            pl.BlockSpec((_SH, _H), lambda i: (0, 0)),
        ],
        out_specs=[pl.BlockSpec((_QB2, _H), lambda i: (i, 0))],
        out_shape=[jax.ShapeDtypeStruct((_BN, _H), _F32)],
    )(gathered, ea_flat, c_q, mask_flat, we)[0]


def _qup_core(hq_ref, agg_ref, na_ref, wut_ref, wub_ref, bu_ref, wn_ref):
    hq = hq_ref[...]
    uq = _silu(jnp.dot(hq, wut_ref[...], precision=_HIGH,
                       preferred_element_type=_F32)
               + jnp.dot(agg_ref[...], wub_ref[...], precision=_HIGH,
                         preferred_element_type=_F32) + bu_ref[...])
    gq = jax.nn.sigmoid(jnp.dot(na_ref[...], wn_ref[...], precision=_HIGH,
                                preferred_element_type=_F32))
    return _rmsnorm(hq + uq * gq)


def _qup_body(hq_ref, agg_ref, na_ref, wut_ref, wub_ref, bu_ref, wn_ref,
              wmb_ref, bm_ref, hqo_ref, cqo_ref):
    hq2 = _qup_core(hq_ref, agg_ref, na_ref, wut_ref, wub_ref, bu_ref, wn_ref)
    hqo_ref[...] = hq2
    cqo_ref[...] = jnp.dot(hq2, wmb_ref[...], precision=_HIGH,
                           preferred_element_type=_F32) + bm_ref[...]


def _qup(h_q, agg, na, wut, wub, bu, wn, wmb_next, bm_next):
    grid = (_BN // _RB,)
    return pl.pallas_call(
        _qup_body,
        grid=grid,
        in_specs=[
            pl.BlockSpec((_RB, _H), lambda i: (i, 0)),
            pl.BlockSpec((_RB, _H), lambda i: (i, 0)),
            pl.BlockSpec((_RB, _SH), lambda i: (i, 0)),
            pl.BlockSpec((_H, _H), lambda i: (0, 0)),
            pl.BlockSpec((_H, _H), lambda i: (0, 0)),
            pl.BlockSpec((1, _H), lambda i: (0, 0)),
            pl.BlockSpec((_SH, _H), lambda i: (0, 0)),
            pl.BlockSpec((_H, _H), lambda i: (0, 0)),
            pl.BlockSpec((1, _H), lambda i: (0, 0)),
        ],
        out_specs=[
            pl.BlockSpec((_RB, _H), lambda i: (i, 0)),
            pl.BlockSpec((_RB, _H), lambda i: (i, 0)),
        ],
        out_shape=[
            jax.ShapeDtypeStruct((_BN, _H), _F32),
            jax.ShapeDtypeStruct((_BN, _H), _F32),
        ],
    )(h_q, agg, na, wut, wub, bu, wn, wmb_next, bm_next)


def _qlast_body(hq_ref, agg_ref, na_ref, wut_ref, wub_ref, bu_ref, wn_ref,
                wo_ref, bo_ref, o_ref):
    hq2 = _qup_core(hq_ref, agg_ref, na_ref, wut_ref, wub_ref, bu_ref, wn_ref)
    o_ref[...] = jnp.dot(hq2, wo_ref[...], precision=_HIGH,
                         preferred_element_type=_F32) + bo_ref[...]


def _qlast(h_q, agg, na, wut, wub, bu, wn, W_out, b_out):
    grid = (_BN // _RB,)
    return pl.pallas_call(
        _qlast_body,
        grid=grid,
        in_specs=[
            pl.BlockSpec((_RB, _H), lambda i: (i, 0)),
            pl.BlockSpec((_RB, _H), lambda i: (i, 0)),
            pl.BlockSpec((_RB, _SH), lambda i: (i, 0)),
            pl.BlockSpec((_H, _H), lambda i: (0, 0)),
            pl.BlockSpec((_H, _H), lambda i: (0, 0)),
            pl.BlockSpec((1, _H), lambda i: (0, 0)),
            pl.BlockSpec((_SH, _H), lambda i: (0, 0)),
            pl.BlockSpec((_H, _NA * 3), lambda i: (0, 0)),
            pl.BlockSpec((1, _NA * 3), lambda i: (0, 0)),
        ],
        out_specs=[pl.BlockSpec((_RB, _NA * 3), lambda i: (i, 0))],
        out_shape=[jax.ShapeDtypeStruct((_BN, _NA * 3), _F32)],
    )(h_q, agg, na, wut, wub, bu, wn, W_out, b_out)[0]


def kernel(query_points, codes, W_embed, b_embed, Wm, bm, We, Wu, bu, Wn,
           W_out, b_out):
    qflat = query_points.reshape(_BN, 3)
    codes_flat = codes.reshape(_BNG, _CODE)

    idxg, ea_flat, maskm, na = _knn(qflat)
    mask_flat = maskm.reshape(_E, 1)
    idx_flat = idxg.reshape(_E)

    h_g, h_q, c_q = _embed(codes_flat, W_embed, b_embed.reshape(1, _H),
                           Wm[0, _H:], bm[0].reshape(1, _H))

    tables = []
    for l in range(_L):
        gp, h_g = _grid_step(h_g, Wm[l, :_H], Wu[l, :_H],
                             bu[l].reshape(1, _H), Wn[l])
        tables.append(gp)

    out = None
    for l in range(_L):
        gathered = _sc_gather(tables[l], idx_flat)
        agg = _edge(gathered, ea_flat, c_q, mask_flat, We[l])
        if l + 1 < _L:
            h_q, c_q = _qup(h_q, agg, na, Wu[l, :_H], Wu[l, _H:],
                            bu[l].reshape(1, _H), Wn[l],
                            Wm[l + 1, _H:], bm[l + 1].reshape(1, _H))
        else:
            out = _qlast(h_q, agg, na, Wu[l, :_H], Wu[l, _H:],
                         bu[l].reshape(1, _H), Wn[l],
                         W_out, b_out.reshape(1, _NA * 3))

    return out.reshape(_B, _N, _NA, 3)


# speedup vs baseline: 5.5655x; 1.0060x over previous
import functools

import jax
import jax.numpy as jnp
from jax import lax
from jax.experimental import pallas as pl
from jax.experimental.pallas import tpu as pltpu
from jax.experimental.pallas import tpu_sc as plsc

_B = 2
_N = 4096
_G = 16
_NG = _G ** 3
_CODE = 128
_K = 32
_H = 240
_L = 3
_SH = 9
_NA = 4
_CUTOFF = 8.0
_SPACING = 1.5
_ORIGIN = -(_G - 1) / 2.0 * _SPACING
_BN = _B * _N
_BNG = _B * _NG
_E = _BN * _K

_HP = 256
_QB = 128
_CAND = 256
_SB = 6
_RB = 512
_QB2 = 128
_EB = _QB2 * _K

_NC = 2
_NS = 16
_NW = _NC * _NS
_CH = 128

_F32 = jnp.float32
_HIGH = lax.Precision.HIGHEST


def _silu(x):
    return x * jax.nn.sigmoid(x)


def _rmsnorm(h):
    return h / jnp.sqrt(jnp.mean(h * h, axis=-1, keepdims=True) + 1e-6)


def _knn_body(q_ref, idx_ref, ea_ref, na_ref):
    qx = q_ref[:, 0:1]
    qy = q_ref[:, 1:2]
    qz = q_ref[:, 2:3]

    j = lax.broadcasted_iota(jnp.int32, (_QB, _CAND), 1)
    jf = j.astype(_F32)
    ox = jnp.floor((jf + 0.5) * (1.0 / 36.0))
    jm = jf - 36.0 * ox
    oy = jnp.floor((jm + 0.5) * (1.0 / 6.0))
    oz = jm - 6.0 * oy

    inv_sp = 1.0 / _SPACING
    bx = jnp.clip(jnp.floor((qx - _ORIGIN) * inv_sp).astype(jnp.int32) - 2, 0, _G - _SB)
    by = jnp.clip(jnp.floor((qy - _ORIGIN) * inv_sp).astype(jnp.int32) - 2, 0, _G - _SB)
    bz = jnp.clip(jnp.floor((qz - _ORIGIN) * inv_sp).astype(jnp.int32) - 2, 0, _G - _SB)

    cx = bx.astype(_F32) + ox
    cy = by.astype(_F32) + oy
    cz = bz.astype(_F32) + oz
    rx = (cx * _SPACING + _ORIGIN) - qx
    ry = (cy * _SPACING + _ORIGIN) - qy
    rz = (cz * _SPACING + _ORIGIN) - qz
    d2 = rx * rx + ry * ry + rz * rz

    valid = j < (_SB * _SB * _SB)
    gidx = (cx * 256.0 + cy * 16.0 + cz).astype(jnp.int32)
    gidx = jnp.where(valid, gidx, 100000 + j)
    big = jnp.float32(3.0e38)
    work = jnp.where(valid, d2, big)

    kiota = lax.broadcasted_iota(jnp.int32, (_QB, _K), 1)
    selg = jnp.zeros((_QB, _K), jnp.int32)
    srx = jnp.zeros((_QB, _K), _F32)
    sry = jnp.zeros((_QB, _K), _F32)
    srz = jnp.zeros((_QB, _K), _F32)
    sd2 = jnp.zeros((_QB, _K), _F32)
    imax = jnp.int32(2 ** 31 - 1)
    for k in range(_K):
        m = jnp.min(work, axis=1, keepdims=True)
        sg = jnp.min(jnp.where(work == m, gidx, imax), axis=1, keepdims=True)
        onehot = gidx == sg
        rx_k = jnp.sum(jnp.where(onehot, rx, 0.0), axis=1, keepdims=True)
        ry_k = jnp.sum(jnp.where(onehot, ry, 0.0), axis=1, keepdims=True)
        rz_k = jnp.sum(jnp.where(onehot, rz, 0.0), axis=1, keepdims=True)
        work = jnp.where(onehot, big, work)
        col = kiota == k
        selg = jnp.where(col, sg, selg)
        srx = jnp.where(col, rx_k, srx)
        sry = jnp.where(col, ry_k, sry)
        srz = jnp.where(col, rz_k, srz)
        sd2 = jnp.where(col, m, sd2)

    dist = jnp.sqrt(sd2 + 1e-12)
    maskv = (dist <= _CUTOFF).astype(_F32)
    x = srx / dist
    y = sry / dist
    z = srz / dist
    c0 = 0.28209479177387814
    c1 = 0.4886025119029199
    c2 = 1.0925484305920792
    c20 = 0.31539156525252005
    c22 = 0.5462742152960396
    comps = [
        jnp.full((_QB, _K), c0, _F32),
        c1 * y, c1 * z, c1 * x,
        c2 * x * y, c2 * y * z, c20 * (3.0 * z * z - 1.0), c2 * x * z,
        c22 * (x * x - y * y)
    ]

    boff = (pl.program_id(0) * _QB // _N) * _NG
    idx_ref[...] = selg + boff
    ea_ref[...] = jnp.stack(comps + [1.0 - maskv], axis=-1).reshape(
        _QB * _K, _SH + 1)
    den = jnp.maximum(jnp.sum(maskv, axis=1, keepdims=True), 1.0)
    na_cols = [jnp.ones((_QB, 1), _F32)]
    for i in range(1, _SH):
        na_cols.append(jnp.sum(comps[i] * maskv, axis=1, keepdims=True) / den)
    na_ref[...] = jnp.concatenate(na_cols, axis=1)


def _knn(qflat):
    grid = (_BN // _QB,)
    return pl.pallas_call(
        _knn_body,
        grid=grid,
        in_specs=[pl.BlockSpec((_QB, 3), lambda i: (i, 0))],
        out_specs=[
            pl.BlockSpec((_QB, _K), lambda i: (i, 0)),
            pl.BlockSpec((_QB * _K, _SH + 1), lambda i: (i, 0)),
            pl.BlockSpec((_QB, _SH), lambda i: (i, 0)),
        ],
        out_shape=[
            jax.ShapeDtypeStruct((_BN, _K), jnp.int32),
            jax.ShapeDtypeStruct((_E, _SH + 1), _F32),
            jax.ShapeDtypeStruct((_BN, _SH), _F32),
        ],
    )(qflat)


def _embed_body(codes_ref, w_ref, b_ref, wmb_ref, bm_ref, hg_ref, hq_ref,
                cq_ref):
    z = jnp.dot(codes_ref[...], w_ref[...], precision=_HIGH,
                preferred_element_type=_F32) + b_ref[...]
    hg_ref[...] = _silu(z)
    hq_row = _silu(b_ref[...])
    hq_ref[...] = jnp.broadcast_to(hq_row, (_RB, _H))
    cq_row = jnp.dot(hq_row, wmb_ref[...], precision=_HIGH,
                     preferred_element_type=_F32) + bm_ref[...]
    cq_ref[...] = jnp.broadcast_to(cq_row, (_RB, _H))


def _embed(codes_flat, W_embed, b_embed, wmb0, bm0):
    grid = (_BNG // _RB,)
    return pl.pallas_call(
        _embed_body,
        grid=grid,
        in_specs=[
            pl.BlockSpec((_RB, _CODE), lambda i: (i, 0)),
            pl.BlockSpec((_CODE, _H), lambda i: (0, 0)),
            pl.BlockSpec((1, _H), lambda i: (0, 0)),
            pl.BlockSpec((_H, _H), lambda i: (0, 0)),
            pl.BlockSpec((1, _H), lambda i: (0, 0)),
        ],
        out_specs=[
            pl.BlockSpec((_RB, _H), lambda i: (i, 0)),
            pl.BlockSpec((_RB, _H), lambda i: (i, 0)),
            pl.BlockSpec((_RB, _H), lambda i: (i, 0)),
        ],
        out_shape=[
            jax.ShapeDtypeStruct((_BNG, _H), _F32),
            jax.ShapeDtypeStruct((_BN, _H), _F32),
            jax.ShapeDtypeStruct((_BN, _H), _F32),
        ],
    )(codes_flat, W_embed, b_embed, wmb0, bm0)


def _grid_body(hg_ref, wmt_ref, wut_ref, bu_ref, wn_ref, gp_ref, hgo_ref):
    hg = hg_ref[...]
    gp = jnp.dot(hg, wmt_ref[...], precision=_HIGH,
                 preferred_element_type=_F32)
    gp_ref[...] = jnp.concatenate(
        [gp, jnp.zeros((_RB, _HP - _H), _F32)], axis=1)
    ug = _silu(jnp.dot(hg, wut_ref[...], precision=_HIGH,
                       preferred_element_type=_F32) + bu_ref[...])
    gg = jax.nn.sigmoid(wn_ref[0:1, :])
    hgo_ref[...] = _rmsnorm(hg + ug * gg)


def _grid_step(h_g, wmt, wut, bu, wn):
    grid = (_BNG // _RB,)
    return pl.pallas_call(
        _grid_body,
        grid=grid,
        in_specs=[
            pl.BlockSpec((_RB, _H), lambda i: (i, 0)),
            pl.BlockSpec((_H, _H), lambda i: (0, 0)),
            pl.BlockSpec((_H, _H), lambda i: (0, 0)),
            pl.BlockSpec((1, _H), lambda i: (0, 0)),
            pl.BlockSpec((_SH, _H), lambda i: (0, 0)),
        ],
        out_specs=[
            pl.BlockSpec((_RB, _HP), lambda i: (i, 0)),
            pl.BlockSpec((_RB, _H), lambda i: (i, 0)),
        ],
        out_shape=[
            jax.ShapeDtypeStruct((_BNG, _HP), _F32),
            jax.ShapeDtypeStruct((_BNG, _H), _F32),
        ],
    )(h_g, wmt, wut, bu, wn)


def _sc_gather_body(table_ref, idx_ref, out_ref, idx_v, rows_v, sem):
    wid = lax.axis_index("s") * _NC + lax.axis_index("c")
    per_w = _E // _NW
    base = wid * per_w

    def chunk(i, carry):
        off = base + i * _CH
        pltpu.sync_copy(idx_ref.at[pl.ds(off, _CH)], idx_v)
        pltpu.async_copy(table_ref.at[idx_v], rows_v, sem).wait()
        pltpu.sync_copy(rows_v, out_ref.at[pl.ds(off, _CH)])
        return carry

    lax.fori_loop(0, per_w // _CH, chunk, 0)


@functools.lru_cache(maxsize=1)
def _sc_gather_fn():
    return pl.kernel(
        _sc_gather_body,
        out_type=jax.ShapeDtypeStruct((_E, _HP), _F32),
        mesh=plsc.VectorSubcoreMesh(core_axis_name="c", subcore_axis_name="s"),
        scratch_types=[
            pltpu.VMEM((_CH,), jnp.int32),
            pltpu.VMEM((_CH, _HP), _F32),
            pltpu.SemaphoreType.DMA,
        ],
    )


def _sc_gather(table, idx):
    return _sc_gather_fn()(table, idx)


def _edge_body(g_ref, ea_ref, cq_ref, we_ref, agg_ref):
    g3 = g_ref[:, :_H].reshape(_QB2, _K, _H)
    pre = g3 + cq_ref[...].reshape(_QB2, 1, _H)
    m = _silu(pre)
    gate = jax.nn.sigmoid(
        jnp.dot(ea_ref[...], we_ref[...], precision=_HIGH,
                preferred_element_type=_F32))
    m = m * gate.reshape(_QB2, _K, _H)
    agg_ref[...] = jnp.sum(m, axis=1)


def _edge(gathered, ea_flat, c_q, we_aug):
    grid = (_BN // _QB2,)
    return pl.pallas_call(
        _edge_body,
        grid=grid,
        in_specs=[
            pl.BlockSpec((_EB, _HP), lambda i: (i, 0)),
            pl.BlockSpec((_EB, _SH + 1), lambda i: (i, 0)),
            pl.BlockSpec((_QB2, _H), lambda i: (i, 0)),
            pl.BlockSpec((_SH + 1, _H), lambda i: (0, 0)),
        ],
        out_specs=[pl.BlockSpec((_QB2, _H), lambda i: (i, 0))],
        out_shape=[jax.ShapeDtypeStruct((_BN, _H), _F32)],
    )(gathered, ea_flat, c_q, we_aug)[0]


def _qup_core(hq_ref, agg_ref, na_ref, wut_ref, wub_ref, bu_ref, wn_ref):
    hq = hq_ref[...]
    uq = _silu(jnp.dot(hq, wut_ref[...], precision=_HIGH,
                       preferred_element_type=_F32)
               + jnp.dot(agg_ref[...], wub_ref[...], precision=_HIGH,
                         preferred_element_type=_F32) + bu_ref[...])
    gq = jax.nn.sigmoid(jnp.dot(na_ref[...], wn_ref[...], precision=_HIGH,
                                preferred_element_type=_F32))
    return _rmsnorm(hq + uq * gq)


def _qup_body(hq_ref, agg_ref, na_ref, wut_ref, wub_ref, bu_ref, wn_ref,
              wmb_ref, bm_ref, hqo_ref, cqo_ref):
    hq2 = _qup_core(hq_ref, agg_ref, na_ref, wut_ref, wub_ref, bu_ref, wn_ref)
    hqo_ref[...] = hq2
    cqo_ref[...] = jnp.dot(hq2, wmb_ref[...], precision=_HIGH,
                           preferred_element_type=_F32) + bm_ref[...]


def _qup(h_q, agg, na, wut, wub, bu, wn, wmb_next, bm_next):
    grid = (_BN // _RB,)
    return pl.pallas_call(
        _qup_body,
        grid=grid,
        in_specs=[
            pl.BlockSpec((_RB, _H), lambda i: (i, 0)),
            pl.BlockSpec((_RB, _H), lambda i: (i, 0)),
            pl.BlockSpec((_RB, _SH), lambda i: (i, 0)),
            pl.BlockSpec((_H, _H), lambda i: (0, 0)),
            pl.BlockSpec((_H, _H), lambda i: (0, 0)),
            pl.BlockSpec((1, _H), lambda i: (0, 0)),
            pl.BlockSpec((_SH, _H), lambda i: (0, 0)),
            pl.BlockSpec((_H, _H), lambda i: (0, 0)),
            pl.BlockSpec((1, _H), lambda i: (0, 0)),
        ],
        out_specs=[
            pl.BlockSpec((_RB, _H), lambda i: (i, 0)),
            pl.BlockSpec((_RB, _H), lambda i: (i, 0)),
        ],
        out_shape=[
            jax.ShapeDtypeStruct((_BN, _H), _F32),
            jax.ShapeDtypeStruct((_BN, _H), _F32),
        ],
    )(h_q, agg, na, wut, wub, bu, wn, wmb_next, bm_next)


def _qlast_body(hq_ref, agg_ref, na_ref, wut_ref, wub_ref, bu_ref, wn_ref,
                wo_ref, bo_ref, o_ref):
    hq2 = _qup_core(hq_ref, agg_ref, na_ref, wut_ref, wub_ref, bu_ref, wn_ref)
    o_ref[...] = jnp.dot(hq2, wo_ref[...], precision=_HIGH,
                         preferred_element_type=_F32) + bo_ref[...]


def _qlast(h_q, agg, na, wut, wub, bu, wn, W_out, b_out):
    grid = (_BN // _RB,)
    return pl.pallas_call(
        _qlast_body,
        grid=grid,
        in_specs=[
            pl.BlockSpec((_RB, _H), lambda i: (i, 0)),
            pl.BlockSpec((_RB, _H), lambda i: (i, 0)),
            pl.BlockSpec((_RB, _SH), lambda i: (i, 0)),
            pl.BlockSpec((_H, _H), lambda i: (0, 0)),
            pl.BlockSpec((_H, _H), lambda i: (0, 0)),
            pl.BlockSpec((1, _H), lambda i: (0, 0)),
            pl.BlockSpec((_SH, _H), lambda i: (0, 0)),
            pl.BlockSpec((_H, _NA * 3), lambda i: (0, 0)),
            pl.BlockSpec((1, _NA * 3), lambda i: (0, 0)),
        ],
        out_specs=[pl.BlockSpec((_RB, _NA * 3), lambda i: (i, 0))],
        out_shape=[jax.ShapeDtypeStruct((_BN, _NA * 3), _F32)],
    )(h_q, agg, na, wut, wub, bu, wn, W_out, b_out)[0]


def kernel(query_points, codes, W_embed, b_embed, Wm, bm, We, Wu, bu, Wn,
           W_out, b_out):
    qflat = query_points.reshape(_BN, 3)
    codes_flat = codes.reshape(_BNG, _CODE)

    idxg, ea_flat, na = _knn(qflat)
    idx_flat = idxg.reshape(_E)
    neg = jnp.full((1, _H), -1e4, _F32)

    h_g, h_q, c_q = _embed(codes_flat, W_embed, b_embed.reshape(1, _H),
                           Wm[0, _H:], bm[0].reshape(1, _H))

    tables = []
    for l in range(_L):
        gp, h_g = _grid_step(h_g, Wm[l, :_H], Wu[l, :_H],
                             bu[l].reshape(1, _H), Wn[l])
        tables.append(gp)

    out = None
    for l in range(_L):
        gathered = _sc_gather(tables[l], idx_flat)
        agg = _edge(gathered, ea_flat, c_q,
                    jnp.concatenate([We[l], neg], axis=0))
        if l + 1 < _L:
            h_q, c_q = _qup(h_q, agg, na, Wu[l, :_H], Wu[l, _H:],
                            bu[l].reshape(1, _H), Wn[l],
                            Wm[l + 1, _H:], bm[l + 1].reshape(1, _H))
        else:
            out = _qlast(h_q, agg, na, Wu[l, :_H], Wu[l, _H:],
                         bu[l].reshape(1, _H), Wn[l],
                         W_out, b_out.reshape(1, _NA * 3))

    return out.reshape(_B, _N, _NA, 3)


# speedup vs baseline: 5.7091x; 1.0258x over previous
import functools

import jax
import jax.numpy as jnp
from jax import lax
from jax.experimental import pallas as pl
from jax.experimental.pallas import tpu as pltpu
from jax.experimental.pallas import tpu_sc as plsc

_B = 2
_N = 4096
_G = 16
_NG = _G ** 3
_CODE = 128
_K = 32
_H = 240
_L = 3
_SH = 9
_NA = 4
_CUTOFF = 8.0
_SPACING = 1.5
_ORIGIN = -(_G - 1) / 2.0 * _SPACING
_BN = _B * _N
_BNG = _B * _NG
_E = _BN * _K

_HP = 256
_QB = 128
_CAND = 128

_OFFS = [
    (-2, -1, 0), (-2, -1, 1), (-2, 0, -1), (-2, 0, 0), (-2, 0, 1),
    (-2, 0, 2), (-2, 1, -1), (-2, 1, 0), (-2, 1, 1), (-2, 1, 2),
    (-2, 2, 0), (-2, 2, 1), (-1, -2, 0), (-1, -2, 1), (-1, -1, -1),
    (-1, -1, 0), (-1, -1, 1), (-1, -1, 2), (-1, 0, -2), (-1, 0, -1),
    (-1, 0, 0), (-1, 0, 1), (-1, 0, 2), (-1, 0, 3), (-1, 1, -2),
    (-1, 1, -1), (-1, 1, 0), (-1, 1, 1), (-1, 1, 2), (-1, 1, 3),
    (-1, 2, -1), (-1, 2, 0), (-1, 2, 1), (-1, 2, 2), (0, -2, 0),
    (0, -2, 1), (0, -1, -1), (0, -1, 0), (0, -1, 1), (0, -1, 2),
    (0, 0, -2), (0, 0, -1), (0, 0, 0), (0, 0, 1), (0, 0, 2), (0, 0, 3),
    (0, 1, -2), (0, 1, -1), (0, 1, 0), (0, 1, 1), (0, 1, 2), (0, 1, 3),
    (0, 2, -1), (0, 2, 0), (0, 2, 1), (0, 2, 2), (0, 3, 0), (0, 3, 1),
    (1, -2, 0), (1, -2, 1), (1, -1, -1), (1, -1, 0), (1, -1, 1),
    (1, -1, 2), (1, 0, -2), (1, 0, -1), (1, 0, 0), (1, 0, 1), (1, 0, 2),
    (1, 0, 3), (1, 1, -2), (1, 1, -1), (1, 1, 0), (1, 1, 1), (1, 1, 2),
    (1, 1, 3), (1, 2, -1), (1, 2, 0), (1, 2, 1), (1, 2, 2), (1, 3, 0),
    (1, 3, 1), (2, -1, -1), (2, -1, 0), (2, -1, 1), (2, -1, 2),
    (2, 0, -1), (2, 0, 0), (2, 0, 1), (2, 0, 2), (2, 1, -1), (2, 1, 0),
    (2, 1, 1), (2, 1, 2), (2, 2, -1), (2, 2, 0), (2, 2, 1), (2, 2, 2),
    (3, 0, 0), (3, 0, 1), (3, 1, 0), (3, 1, 1),
]
_NOFF = len(_OFFS)
_RB = 512
_QB2 = 128
_EB = _QB2 * _K

_NC = 2
_NS = 16
_NW = _NC * _NS
_CH = 128

_F32 = jnp.float32
_HIGH = lax.Precision.HIGHEST


def _silu(x):
    return x * jax.nn.sigmoid(x)


def _rmsnorm(h):
    return h / jnp.sqrt(jnp.mean(h * h, axis=-1, keepdims=True) + 1e-6)


def _knn_body(q_ref, offs_ref, idx_ref, ea_ref, na_ref):
    qx = q_ref[:, 0:1]
    qy = q_ref[:, 1:2]
    qz = q_ref[:, 2:3]

    j = lax.broadcasted_iota(jnp.int32, (_QB, _CAND), 1)
    ox = offs_ref[0:1, :]
    oy = offs_ref[1:2, :]
    oz = offs_ref[2:3, :]

    inv_sp = 1.0 / _SPACING
    fx = jnp.floor((qx - _ORIGIN) * inv_sp)
    fy = jnp.floor((qy - _ORIGIN) * inv_sp)
    fz = jnp.floor((qz - _ORIGIN) * inv_sp)

    cx = jnp.clip(fx + ox, 0.0, _G - 1.0)
    cy = jnp.clip(fy + oy, 0.0, _G - 1.0)
    cz = jnp.clip(fz + oz, 0.0, _G - 1.0)
    rx = (cx * _SPACING + _ORIGIN) - qx
    ry = (cy * _SPACING + _ORIGIN) - qy
    rz = (cz * _SPACING + _ORIGIN) - qz
    d2 = rx * rx + ry * ry + rz * rz

    valid = j < _NOFF
    gidx = (cx * 256.0 + cy * 16.0 + cz).astype(jnp.int32)
    gidx = jnp.where(valid, gidx, 100000 + j)
    big = jnp.float32(3.0e38)
    work = jnp.where(valid, d2, big)

    kiota = lax.broadcasted_iota(jnp.int32, (_QB, _K), 1)
    selg = jnp.zeros((_QB, _K), jnp.int32)
    imax = jnp.int32(2 ** 31 - 1)
    for k in range(_K):
        m = jnp.min(work, axis=1, keepdims=True)
        sg = jnp.min(jnp.where(work == m, gidx, imax), axis=1, keepdims=True)
        work = jnp.where(gidx == sg, big, work)
        selg = jnp.where(kiota == k, sg, selg)

    scx = lax.shift_right_logical(selg, 8).astype(_F32)
    scy = jnp.bitwise_and(lax.shift_right_logical(selg, 4), 15).astype(_F32)
    scz = jnp.bitwise_and(selg, 15).astype(_F32)
    srx = (scx * _SPACING + _ORIGIN) - qx
    sry = (scy * _SPACING + _ORIGIN) - qy
    srz = (scz * _SPACING + _ORIGIN) - qz
    sd2 = srx * srx + sry * sry + srz * srz

    dist = jnp.sqrt(sd2 + 1e-12)
    maskv = (dist <= _CUTOFF).astype(_F32)
    x = srx / dist
    y = sry / dist
    z = srz / dist
    c0 = 0.28209479177387814
    c1 = 0.4886025119029199
    c2 = 1.0925484305920792
    c20 = 0.31539156525252005
    c22 = 0.5462742152960396
    comps = [
        jnp.full((_QB, _K), c0, _F32),
        c1 * y, c1 * z, c1 * x,
        c2 * x * y, c2 * y * z, c20 * (3.0 * z * z - 1.0), c2 * x * z,
        c22 * (x * x - y * y)
    ]

    boff = (pl.program_id(0) * _QB // _N) * _NG
    idx_ref[...] = selg + boff
    ea_ref[...] = jnp.stack(comps + [1.0 - maskv], axis=-1).reshape(
        _QB * _K, _SH + 1)
    den = jnp.maximum(jnp.sum(maskv, axis=1, keepdims=True), 1.0)
    na_cols = [jnp.ones((_QB, 1), _F32)]
    for i in range(1, _SH):
        na_cols.append(jnp.sum(comps[i] * maskv, axis=1, keepdims=True) / den)
    na_ref[...] = jnp.concatenate(na_cols, axis=1)


def _knn(qflat):
    pad = (0,) * (_CAND - _NOFF)
    offs = jnp.asarray(
        [[float(o[d]) for o in _OFFS] + list(pad) for d in range(3)], _F32)
    grid = (_BN // _QB,)
    return pl.pallas_call(
        _knn_body,
        grid=grid,
        in_specs=[pl.BlockSpec((_QB, 3), lambda i: (i, 0)),
                  pl.BlockSpec((3, _CAND), lambda i: (0, 0))],
        out_specs=[
            pl.BlockSpec((_QB, _K), lambda i: (i, 0)),
            pl.BlockSpec((_QB * _K, _SH + 1), lambda i: (i, 0)),
            pl.BlockSpec((_QB, _SH), lambda i: (i, 0)),
        ],
        out_shape=[
            jax.ShapeDtypeStruct((_BN, _K), jnp.int32),
            jax.ShapeDtypeStruct((_E, _SH + 1), _F32),
            jax.ShapeDtypeStruct((_BN, _SH), _F32),
        ],
    )(qflat, offs)


def _embed_body(codes_ref, w_ref, b_ref, wmb_ref, bm_ref, hg_ref, hq_ref,
                cq_ref):
    z = jnp.dot(codes_ref[...], w_ref[...], precision=_HIGH,
                preferred_element_type=_F32) + b_ref[...]
    hg_ref[...] = _silu(z)
    hq_row = _silu(b_ref[...])
    hq_ref[...] = jnp.broadcast_to(hq_row, (_RB, _H))
    cq_row = jnp.dot(hq_row, wmb_ref[...], precision=_HIGH,
                     preferred_element_type=_F32) + bm_ref[...]
    cq_ref[...] = jnp.broadcast_to(cq_row, (_RB, _H))


def _embed(codes_flat, W_embed, b_embed, wmb0, bm0):
    grid = (_BNG // _RB,)
    return pl.pallas_call(
        _embed_body,
        grid=grid,
        in_specs=[
            pl.BlockSpec((_RB, _CODE), lambda i: (i, 0)),
            pl.BlockSpec((_CODE, _H), lambda i: (0, 0)),
            pl.BlockSpec((1, _H), lambda i: (0, 0)),
            pl.BlockSpec((_H, _H), lambda i: (0, 0)),
            pl.BlockSpec((1, _H), lambda i: (0, 0)),
        ],
        out_specs=[
            pl.BlockSpec((_RB, _H), lambda i: (i, 0)),
            pl.BlockSpec((_RB, _H), lambda i: (i, 0)),
            pl.BlockSpec((_RB, _H), lambda i: (i, 0)),
        ],
        out_shape=[
            jax.ShapeDtypeStruct((_BNG, _H), _F32),
            jax.ShapeDtypeStruct((_BN, _H), _F32),
            jax.ShapeDtypeStruct((_BN, _H), _F32),
        ],
    )(codes_flat, W_embed, b_embed, wmb0, bm0)


def _grid_body(hg_ref, wmt_ref, wut_ref, bu_ref, wn_ref, gp_ref, hgo_ref):
    hg = hg_ref[...]
    gp = jnp.dot(hg, wmt_ref[...], precision=_HIGH,
                 preferred_element_type=_F32)
    gp_ref[...] = jnp.concatenate(
        [gp, jnp.zeros((_RB, _HP - _H), _F32)], axis=1)
    ug = _silu(jnp.dot(hg, wut_ref[...], precision=_HIGH,
                       preferred_element_type=_F32) + bu_ref[...])
    gg = jax.nn.sigmoid(wn_ref[0:1, :])
    hgo_ref[...] = _rmsnorm(hg + ug * gg)


def _grid_step(h_g, wmt, wut, bu, wn):
    grid = (_BNG // _RB,)
    return pl.pallas_call(
        _grid_body,
        grid=grid,
        in_specs=[
            pl.BlockSpec((_RB, _H), lambda i: (i, 0)),
            pl.BlockSpec((_H, _H), lambda i: (0, 0)),
            pl.BlockSpec((_H, _H), lambda i: (0, 0)),
            pl.BlockSpec((1, _H), lambda i: (0, 0)),
            pl.BlockSpec((_SH, _H), lambda i: (0, 0)),
        ],
        out_specs=[
            pl.BlockSpec((_RB, _HP), lambda i: (i, 0)),
            pl.BlockSpec((_RB, _H), lambda i: (i, 0)),
        ],
        out_shape=[
            jax.ShapeDtypeStruct((_BNG, _HP), _F32),
            jax.ShapeDtypeStruct((_BNG, _H), _F32),
        ],
    )(h_g, wmt, wut, bu, wn)


_GRP = 8


def _sc_gather_body(table_ref, idx_ref, out_ref, idx_v, rows0, rows1, sem0,
                    sem1):
    wid = lax.axis_index("s") * _NC + lax.axis_index("c")
    per_w = _E // _NW
    base = wid * per_w
    pltpu.sync_copy(idx_ref.at[pl.ds(base, per_w)], idx_v)
    bufs = (rows0, rows1)
    sems = (sem0, sem1)

    def group(g, carry):
        cps = []
        for jj in range(_GRP):
            i = g * _GRP + jj
            cp = pltpu.async_copy(
                table_ref.at[idx_v.at[pl.ds(i * _CH, _CH)]],
                bufs[jj % 2], sems[jj % 2])
            cps.append(cp)
            if jj >= 1:
                cps[jj - 1].wait()
                pltpu.sync_copy(
                    bufs[(jj - 1) % 2],
                    out_ref.at[pl.ds(base + (i - 1) * _CH, _CH)])
        cps[_GRP - 1].wait()
        pltpu.sync_copy(
            bufs[(_GRP - 1) % 2],
            out_ref.at[pl.ds(base + (g * _GRP + _GRP - 1) * _CH, _CH)])
        return carry

    lax.fori_loop(0, per_w // _CH // _GRP, group, 0)


@functools.lru_cache(maxsize=1)
def _sc_gather_fn():
    return pl.kernel(
        _sc_gather_body,
        out_type=jax.ShapeDtypeStruct((_E, _HP), _F32),
        mesh=plsc.VectorSubcoreMesh(core_axis_name="c", subcore_axis_name="s"),
        scratch_types=[
            pltpu.VMEM((_E // _NW,), jnp.int32),
            pltpu.VMEM((_CH, _HP), _F32),
            pltpu.VMEM((_CH, _HP), _F32),
            pltpu.SemaphoreType.DMA,
            pltpu.SemaphoreType.DMA,
        ],
    )


def _sc_gather(table, idx):
    return _sc_gather_fn()(table, idx)


def _edge_body(g_ref, ea_ref, cq_ref, we_ref, agg_ref):
    g3 = g_ref[:, :_H].reshape(_QB2, _K, _H)
    pre = g3 + cq_ref[...].reshape(_QB2, 1, _H)
    m = _silu(pre)
    gate = jax.nn.sigmoid(
        jnp.dot(ea_ref[...], we_ref[...], precision=_HIGH,
                preferred_element_type=_F32))
    m = m * gate.reshape(_QB2, _K, _H)
    agg_ref[...] = jnp.sum(m, axis=1)


def _edge(gathered, ea_flat, c_q, we_aug):
    grid = (_BN // _QB2,)
    return pl.pallas_call(
        _edge_body,
        grid=grid,
        in_specs=[
            pl.BlockSpec((_EB, _HP), lambda i: (i, 0)),
            pl.BlockSpec((_EB, _SH + 1), lambda i: (i, 0)),
            pl.BlockSpec((_QB2, _H), lambda i: (i, 0)),
            pl.BlockSpec((_SH + 1, _H), lambda i: (0, 0)),
        ],
        out_specs=[pl.BlockSpec((_QB2, _H), lambda i: (i, 0))],
        out_shape=[jax.ShapeDtypeStruct((_BN, _H), _F32)],
    )(gathered, ea_flat, c_q, we_aug)[0]


def _qup_core(hq_ref, agg_ref, na_ref, wut_ref, wub_ref, bu_ref, wn_ref):
    hq = hq_ref[...]
    uq = _silu(jnp.dot(hq, wut_ref[...], precision=_HIGH,
                       preferred_element_type=_F32)
               + jnp.dot(agg_ref[...], wub_ref[...], precision=_HIGH,
                         preferred_element_type=_F32) + bu_ref[...])
    gq = jax.nn.sigmoid(jnp.dot(na_ref[...], wn_ref[...], precision=_HIGH,
                                preferred_element_type=_F32))
    return _rmsnorm(hq + uq * gq)


def _qup_body(hq_ref, agg_ref, na_ref, wut_ref, wub_ref, bu_ref, wn_ref,
              wmb_ref, bm_ref, hqo_ref, cqo_ref):
    hq2 = _qup_core(hq_ref, agg_ref, na_ref, wut_ref, wub_ref, bu_ref, wn_ref)
    hqo_ref[...] = hq2
    cqo_ref[...] = jnp.dot(hq2, wmb_ref[...], precision=_HIGH,
                           preferred_element_type=_F32) + bm_ref[...]


def _qup(h_q, agg, na, wut, wub, bu, wn, wmb_next, bm_next):
    grid = (_BN // _RB,)
    return pl.pallas_call(
        _qup_body,
        grid=grid,
        in_specs=[
            pl.BlockSpec((_RB, _H), lambda i: (i, 0)),
            pl.BlockSpec((_RB, _H), lambda i: (i, 0)),
            pl.BlockSpec((_RB, _SH), lambda i: (i, 0)),
            pl.BlockSpec((_H, _H), lambda i: (0, 0)),
            pl.BlockSpec((_H, _H), lambda i: (0, 0)),
            pl.BlockSpec((1, _H), lambda i: (0, 0)),
            pl.BlockSpec((_SH, _H), lambda i: (0, 0)),
            pl.BlockSpec((_H, _H), lambda i: (0, 0)),
            pl.BlockSpec((1, _H), lambda i: (0, 0)),
        ],
        out_specs=[
            pl.BlockSpec((_RB, _H), lambda i: (i, 0)),
            pl.BlockSpec((_RB, _H), lambda i: (i, 0)),
        ],
        out_shape=[
            jax.ShapeDtypeStruct((_BN, _H), _F32),
            jax.ShapeDtypeStruct((_BN, _H), _F32),
        ],
    )(h_q, agg, na, wut, wub, bu, wn, wmb_next, bm_next)


def _qlast_body(hq_ref, agg_ref, na_ref, wut_ref, wub_ref, bu_ref, wn_ref,
                wo_ref, bo_ref, o_ref):
    hq2 = _qup_core(hq_ref, agg_ref, na_ref, wut_ref, wub_ref, bu_ref, wn_ref)
    o_ref[...] = jnp.dot(hq2, wo_ref[...], precision=_HIGH,
                         preferred_element_type=_F32) + bo_ref[...]


def _qlast(h_q, agg, na, wut, wub, bu, wn, W_out, b_out):
    grid = (_BN // _RB,)
    return pl.pallas_call(
        _qlast_body,
        grid=grid,
        in_specs=[
            pl.BlockSpec((_RB, _H), lambda i: (i, 0)),
            pl.BlockSpec((_RB, _H), lambda i: (i, 0)),
            pl.BlockSpec((_RB, _SH), lambda i: (i, 0)),
            pl.BlockSpec((_H, _H), lambda i: (0, 0)),
            pl.BlockSpec((_H, _H), lambda i: (0, 0)),
            pl.BlockSpec((1, _H), lambda i: (0, 0)),
            pl.BlockSpec((_SH, _H), lambda i: (0, 0)),
            pl.BlockSpec((_H, _NA * 3), lambda i: (0, 0)),
            pl.BlockSpec((1, _NA * 3), lambda i: (0, 0)),
        ],
        out_specs=[pl.BlockSpec((_RB, _NA * 3), lambda i: (i, 0))],
        out_shape=[jax.ShapeDtypeStruct((_BN, _NA * 3), _F32)],
    )(h_q, agg, na, wut, wub, bu, wn, W_out, b_out)[0]


def kernel(query_points, codes, W_embed, b_embed, Wm, bm, We, Wu, bu, Wn,
           W_out, b_out):
    qflat = query_points.reshape(_BN, 3)
    codes_flat = codes.reshape(_BNG, _CODE)

    idxg, ea_flat, na = _knn(qflat)
    idx_flat = idxg.reshape(_E)
    neg = jnp.full((1, _H), -1e4, _F32)

    h_g, h_q, c_q = _embed(codes_flat, W_embed, b_embed.reshape(1, _H),
                           Wm[0, _H:], bm[0].reshape(1, _H))

    tables = []
    for l in range(_L):
        gp, h_g = _grid_step(h_g, Wm[l, :_H], Wu[l, :_H],
                             bu[l].reshape(1, _H), Wn[l])
        tables.append(gp)

    out = None
    for l in range(_L):
        gathered = _sc_gather(tables[l], idx_flat)
        agg = _edge(gathered, ea_flat, c_q,
                    jnp.concatenate([We[l], neg], axis=0))
        if l + 1 < _L:
            h_q, c_q = _qup(h_q, agg, na, Wu[l, :_H], Wu[l, _H:],
                            bu[l].reshape(1, _H), Wn[l],
                            Wm[l + 1, _H:], bm[l + 1].reshape(1, _H))
        else:
            out = _qlast(h_q, agg, na, Wu[l, :_H], Wu[l, _H:],
                         bu[l].reshape(1, _H), Wn[l],
                         W_out, b_out.reshape(1, _NA * 3))

    return out.reshape(_B, _N, _NA, 3)


# speedup vs baseline: 6.2340x; 1.0919x over previous
import functools

import jax
import jax.numpy as jnp
from jax import lax
from jax.experimental import pallas as pl
from jax.experimental.pallas import tpu as pltpu
from jax.experimental.pallas import tpu_sc as plsc

_B = 2
_N = 4096
_G = 16
_NG = _G ** 3
_CODE = 128
_K = 32
_H = 240
_L = 3
_SH = 9
_NA = 4
_CUTOFF = 8.0
_SPACING = 1.5
_ORIGIN = -(_G - 1) / 2.0 * _SPACING
_BN = _B * _N
_BNG = _B * _NG
_E = _BN * _K

_HP = 256
_QB = 128
_CAND = 128

_OFFS = [
    (-2, -1, 0), (-2, -1, 1), (-2, 0, -1), (-2, 0, 0), (-2, 0, 1),
    (-2, 0, 2), (-2, 1, -1), (-2, 1, 0), (-2, 1, 1), (-2, 1, 2),
    (-2, 2, 0), (-2, 2, 1), (-1, -2, 0), (-1, -2, 1), (-1, -1, -1),
    (-1, -1, 0), (-1, -1, 1), (-1, -1, 2), (-1, 0, -2), (-1, 0, -1),
    (-1, 0, 0), (-1, 0, 1), (-1, 0, 2), (-1, 0, 3), (-1, 1, -2),
    (-1, 1, -1), (-1, 1, 0), (-1, 1, 1), (-1, 1, 2), (-1, 1, 3),
    (-1, 2, -1), (-1, 2, 0), (-1, 2, 1), (-1, 2, 2), (0, -2, 0),
    (0, -2, 1), (0, -1, -1), (0, -1, 0), (0, -1, 1), (0, -1, 2),
    (0, 0, -2), (0, 0, -1), (0, 0, 0), (0, 0, 1), (0, 0, 2), (0, 0, 3),
    (0, 1, -2), (0, 1, -1), (0, 1, 0), (0, 1, 1), (0, 1, 2), (0, 1, 3),
    (0, 2, -1), (0, 2, 0), (0, 2, 1), (0, 2, 2), (0, 3, 0), (0, 3, 1),
    (1, -2, 0), (1, -2, 1), (1, -1, -1), (1, -1, 0), (1, -1, 1),
    (1, -1, 2), (1, 0, -2), (1, 0, -1), (1, 0, 0), (1, 0, 1), (1, 0, 2),
    (1, 0, 3), (1, 1, -2), (1, 1, -1), (1, 1, 0), (1, 1, 1), (1, 1, 2),
    (1, 1, 3), (1, 2, -1), (1, 2, 0), (1, 2, 1), (1, 2, 2), (1, 3, 0),
    (1, 3, 1), (2, -1, -1), (2, -1, 0), (2, -1, 1), (2, -1, 2),
    (2, 0, -1), (2, 0, 0), (2, 0, 1), (2, 0, 2), (2, 1, -1), (2, 1, 0),
    (2, 1, 1), (2, 1, 2), (2, 2, -1), (2, 2, 0), (2, 2, 1), (2, 2, 2),
    (3, 0, 0), (3, 0, 1), (3, 1, 0), (3, 1, 1),
]
_NOFF = len(_OFFS)
_RB = 512
_QB2 = 128
_EB = _QB2 * _K

_NC = 2
_NS = 16
_NW = _NC * _NS
_CH = 128

_F32 = jnp.float32
_HIGH = lax.Precision.HIGHEST


def _silu(x):
    return x * jax.nn.sigmoid(x)


def _rmsnorm(h):
    return h / jnp.sqrt(jnp.mean(h * h, axis=-1, keepdims=True) + 1e-6)


def _knn_body(q_ref, offs_ref, idx_ref, ea_ref, na_ref):
    qx = q_ref[:, 0:1]
    qy = q_ref[:, 1:2]
    qz = q_ref[:, 2:3]

    j = lax.broadcasted_iota(jnp.int32, (_QB, _CAND), 1)
    ox = offs_ref[0:1, :]
    oy = offs_ref[1:2, :]
    oz = offs_ref[2:3, :]

    inv_sp = 1.0 / _SPACING
    fx = jnp.floor((qx - _ORIGIN) * inv_sp)
    fy = jnp.floor((qy - _ORIGIN) * inv_sp)
    fz = jnp.floor((qz - _ORIGIN) * inv_sp)

    cx = jnp.clip(fx + ox, 0.0, _G - 1.0)
    cy = jnp.clip(fy + oy, 0.0, _G - 1.0)
    cz = jnp.clip(fz + oz, 0.0, _G - 1.0)
    rx = (cx * _SPACING + _ORIGIN) - qx
    ry = (cy * _SPACING + _ORIGIN) - qy
    rz = (cz * _SPACING + _ORIGIN) - qz
    d2 = rx * rx + ry * ry + rz * rz

    valid = j < _NOFF
    gidx = (cx * 256.0 + cy * 16.0 + cz).astype(jnp.int32)
    gidx = jnp.where(valid, gidx, 100000 + j)
    big = jnp.float32(3.0e38)
    work = jnp.where(valid, d2, big)

    kiota = lax.broadcasted_iota(jnp.int32, (_QB, _K), 1)
    selg = jnp.zeros((_QB, _K), jnp.int32)
    imax = jnp.int32(2 ** 31 - 1)
    for k in range(_K):
        m = jnp.min(work, axis=1, keepdims=True)
        sg = jnp.min(jnp.where(work == m, gidx, imax), axis=1, keepdims=True)
        work = jnp.where(gidx == sg, big, work)
        selg = jnp.where(kiota == k, sg, selg)

    scx = lax.shift_right_logical(selg, 8).astype(_F32)
    scy = jnp.bitwise_and(lax.shift_right_logical(selg, 4), 15).astype(_F32)
    scz = jnp.bitwise_and(selg, 15).astype(_F32)
    srx = (scx * _SPACING + _ORIGIN) - qx
    sry = (scy * _SPACING + _ORIGIN) - qy
    srz = (scz * _SPACING + _ORIGIN) - qz
    sd2 = srx * srx + sry * sry + srz * srz

    dist = jnp.sqrt(sd2 + 1e-12)
    maskv = (dist <= _CUTOFF).astype(_F32)
    x = srx / dist
    y = sry / dist
    z = srz / dist
    c0 = 0.28209479177387814
    c1 = 0.4886025119029199
    c2 = 1.0925484305920792
    c20 = 0.31539156525252005
    c22 = 0.5462742152960396
    comps = [
        jnp.full((_QB, _K), c0, _F32),
        c1 * y, c1 * z, c1 * x,
        c2 * x * y, c2 * y * z, c20 * (3.0 * z * z - 1.0), c2 * x * z,
        c22 * (x * x - y * y)
    ]

    boff = (pl.program_id(0) * _QB // _N) * _NG
    idx_ref[...] = selg + boff
    ea_ref[...] = jnp.swapaxes(
        jnp.stack(comps + [1.0 - maskv], axis=1), 1, 2).reshape(
            _QB * _K, _SH + 1)
    den = jnp.maximum(jnp.sum(maskv, axis=1, keepdims=True), 1.0)
    na_cols = [jnp.ones((_QB, 1), _F32)]
    for i in range(1, _SH):
        na_cols.append(jnp.sum(comps[i] * maskv, axis=1, keepdims=True) / den)
    na_ref[...] = jnp.concatenate(na_cols, axis=1)


def _knn(qflat):
    pad = (0,) * (_CAND - _NOFF)
    offs = jnp.asarray(
        [[float(o[d]) for o in _OFFS] + list(pad) for d in range(3)], _F32)
    grid = (_BN // _QB,)
    return pl.pallas_call(
        _knn_body,
        grid=grid,
        in_specs=[pl.BlockSpec((_QB, 3), lambda i: (i, 0)),
                  pl.BlockSpec((3, _CAND), lambda i: (0, 0))],
        out_specs=[
            pl.BlockSpec((_QB, _K), lambda i: (i, 0)),
            pl.BlockSpec((_QB * _K, _SH + 1), lambda i: (i, 0)),
            pl.BlockSpec((_QB, _SH), lambda i: (i, 0)),
        ],
        out_shape=[
            jax.ShapeDtypeStruct((_BN, _K), jnp.int32),
            jax.ShapeDtypeStruct((_E, _SH + 1), _F32),
            jax.ShapeDtypeStruct((_BN, _SH), _F32),
        ],
    )(qflat, offs)


def _embed_body(codes_ref, w_ref, b_ref, wmb_ref, bm_ref, hg_ref, hq_ref,
                cq_ref):
    z = jnp.dot(codes_ref[...], w_ref[...], precision=_HIGH,
                preferred_element_type=_F32) + b_ref[...]
    hg_ref[...] = _silu(z)
    hq_row = _silu(b_ref[...])
    hq_ref[...] = jnp.broadcast_to(hq_row, (_RB, _H))
    cq_row = jnp.dot(hq_row, wmb_ref[...], precision=_HIGH,
                     preferred_element_type=_F32) + bm_ref[...]
    cq_ref[...] = jnp.broadcast_to(cq_row, (_RB, _H))


def _embed(codes_flat, W_embed, b_embed, wmb0, bm0):
    grid = (_BNG // _RB,)
    return pl.pallas_call(
        _embed_body,
        grid=grid,
        in_specs=[
            pl.BlockSpec((_RB, _CODE), lambda i: (i, 0)),
            pl.BlockSpec((_CODE, _H), lambda i: (0, 0)),
            pl.BlockSpec((1, _H), lambda i: (0, 0)),
            pl.BlockSpec((_H, _H), lambda i: (0, 0)),
            pl.BlockSpec((1, _H), lambda i: (0, 0)),
        ],
        out_specs=[
            pl.BlockSpec((_RB, _H), lambda i: (i, 0)),
            pl.BlockSpec((_RB, _H), lambda i: (i, 0)),
            pl.BlockSpec((_RB, _H), lambda i: (i, 0)),
        ],
        out_shape=[
            jax.ShapeDtypeStruct((_BNG, _H), _F32),
            jax.ShapeDtypeStruct((_BN, _H), _F32),
            jax.ShapeDtypeStruct((_BN, _H), _F32),
        ],
    )(codes_flat, W_embed, b_embed, wmb0, bm0)


def _grid_body(hg_ref, wmt_ref, wut_ref, bu_ref, wn_ref, gp_ref, hgo_ref):
    hg = hg_ref[...]
    gp = jnp.dot(hg, wmt_ref[...], precision=_HIGH,
                 preferred_element_type=_F32)
    gp_ref[...] = jnp.concatenate(
        [gp, jnp.zeros((_RB, _HP - _H), _F32)], axis=1)
    ug = _silu(jnp.dot(hg, wut_ref[...], precision=_HIGH,
                       preferred_element_type=_F32) + bu_ref[...])
    gg = jax.nn.sigmoid(wn_ref[0:1, :])
    hgo_ref[...] = _rmsnorm(hg + ug * gg)


def _grid_step(h_g, wmt, wut, bu, wn):
    grid = (_BNG // _RB,)
    return pl.pallas_call(
        _grid_body,
        grid=grid,
        in_specs=[
            pl.BlockSpec((_RB, _H), lambda i: (i, 0)),
            pl.BlockSpec((_H, _H), lambda i: (0, 0)),
            pl.BlockSpec((_H, _H), lambda i: (0, 0)),
            pl.BlockSpec((1, _H), lambda i: (0, 0)),
            pl.BlockSpec((_SH, _H), lambda i: (0, 0)),
        ],
        out_specs=[
            pl.BlockSpec((_RB, _HP), lambda i: (i, 0)),
            pl.BlockSpec((_RB, _H), lambda i: (i, 0)),
        ],
        out_shape=[
            jax.ShapeDtypeStruct((_BNG, _HP), _F32),
            jax.ShapeDtypeStruct((_BNG, _H), _F32),
        ],
    )(h_g, wmt, wut, bu, wn)


_GRP = 8


def _sc_gather_body(table_ref, idx_ref, out_ref, idx_v, rows0, rows1, sem0,
                    sem1):
    wid = lax.axis_index("s") * _NC + lax.axis_index("c")
    per_w = _E // _NW
    base = wid * per_w
    pltpu.sync_copy(idx_ref.at[pl.ds(base, per_w)], idx_v)
    bufs = (rows0, rows1)
    sems = (sem0, sem1)

    def group(g, carry):
        cps = []
        for jj in range(_GRP):
            i = g * _GRP + jj
            cp = pltpu.async_copy(
                table_ref.at[idx_v.at[pl.ds(i * _CH, _CH)]],
                bufs[jj % 2], sems[jj % 2])
            cps.append(cp)
            if jj >= 1:
                cps[jj - 1].wait()
                pltpu.sync_copy(
                    bufs[(jj - 1) % 2],
                    out_ref.at[pl.ds(base + (i - 1) * _CH, _CH)])
        cps[_GRP - 1].wait()
        pltpu.sync_copy(
            bufs[(_GRP - 1) % 2],
            out_ref.at[pl.ds(base + (g * _GRP + _GRP - 1) * _CH, _CH)])
        return carry

    lax.fori_loop(0, per_w // _CH // _GRP, group, 0)


@functools.lru_cache(maxsize=1)
def _sc_gather_fn():
    return pl.kernel(
        _sc_gather_body,
        out_type=jax.ShapeDtypeStruct((_E, _HP), _F32),
        mesh=plsc.VectorSubcoreMesh(core_axis_name="c", subcore_axis_name="s"),
        scratch_types=[
            pltpu.VMEM((_E // _NW,), jnp.int32),
            pltpu.VMEM((_CH, _HP), _F32),
            pltpu.VMEM((_CH, _HP), _F32),
            pltpu.SemaphoreType.DMA,
            pltpu.SemaphoreType.DMA,
        ],
    )


def _sc_gather(table, idx):
    return _sc_gather_fn()(table, idx)


def _edge_body(g_ref, ea_ref, cq_ref, we_ref, agg_ref):
    g3 = g_ref[:, :_H].reshape(_QB2, _K, _H)
    pre = g3 + cq_ref[...].reshape(_QB2, 1, _H)
    m = _silu(pre)
    gate = jax.nn.sigmoid(
        jnp.dot(ea_ref[...], we_ref[...], precision=_HIGH,
                preferred_element_type=_F32))
    m = m * gate.reshape(_QB2, _K, _H)
    agg_ref[...] = jnp.sum(m, axis=1)


def _edge(gathered, ea_flat, c_q, we_aug):
    grid = (_BN // _QB2,)
    return pl.pallas_call(
        _edge_body,
        grid=grid,
        in_specs=[
            pl.BlockSpec((_EB, _HP), lambda i: (i, 0)),
            pl.BlockSpec((_EB, _SH + 1), lambda i: (i, 0)),
            pl.BlockSpec((_QB2, _H), lambda i: (i, 0)),
            pl.BlockSpec((_SH + 1, _H), lambda i: (0, 0)),
        ],
        out_specs=[pl.BlockSpec((_QB2, _H), lambda i: (i, 0))],
        out_shape=[jax.ShapeDtypeStruct((_BN, _H), _F32)],
    )(gathered, ea_flat, c_q, we_aug)[0]


def _qup_core(hq_ref, agg_ref, na_ref, wut_ref, wub_ref, bu_ref, wn_ref):
    hq = hq_ref[...]
    uq = _silu(jnp.dot(hq, wut_ref[...], precision=_HIGH,
                       preferred_element_type=_F32)
               + jnp.dot(agg_ref[...], wub_ref[...], precision=_HIGH,
                         preferred_element_type=_F32) + bu_ref[...])
    gq = jax.nn.sigmoid(jnp.dot(na_ref[...], wn_ref[...], precision=_HIGH,
                                preferred_element_type=_F32))
    return _rmsnorm(hq + uq * gq)


def _qup_body(hq_ref, agg_ref, na_ref, wut_ref, wub_ref, bu_ref, wn_ref,
              wmb_ref, bm_ref, hqo_ref, cqo_ref):
    hq2 = _qup_core(hq_ref, agg_ref, na_ref, wut_ref, wub_ref, bu_ref, wn_ref)
    hqo_ref[...] = hq2
    cqo_ref[...] = jnp.dot(hq2, wmb_ref[...], precision=_HIGH,
                           preferred_element_type=_F32) + bm_ref[...]


def _qup(h_q, agg, na, wut, wub, bu, wn, wmb_next, bm_next):
    grid = (_BN // _RB,)
    return pl.pallas_call(
        _qup_body,
        grid=grid,
        in_specs=[
            pl.BlockSpec((_RB, _H), lambda i: (i, 0)),
            pl.BlockSpec((_RB, _H), lambda i: (i, 0)),
            pl.BlockSpec((_RB, _SH), lambda i: (i, 0)),
            pl.BlockSpec((_H, _H), lambda i: (0, 0)),
            pl.BlockSpec((_H, _H), lambda i: (0, 0)),
            pl.BlockSpec((1, _H), lambda i: (0, 0)),
            pl.BlockSpec((_SH, _H), lambda i: (0, 0)),
            pl.BlockSpec((_H, _H), lambda i: (0, 0)),
            pl.BlockSpec((1, _H), lambda i: (0, 0)),
        ],
        out_specs=[
            pl.BlockSpec((_RB, _H), lambda i: (i, 0)),
            pl.BlockSpec((_RB, _H), lambda i: (i, 0)),
        ],
        out_shape=[
            jax.ShapeDtypeStruct((_BN, _H), _F32),
            jax.ShapeDtypeStruct((_BN, _H), _F32),
        ],
    )(h_q, agg, na, wut, wub, bu, wn, wmb_next, bm_next)


def _qlast_body(hq_ref, agg_ref, na_ref, wut_ref, wub_ref, bu_ref, wn_ref,
                wo_ref, bo_ref, o_ref):
    hq2 = _qup_core(hq_ref, agg_ref, na_ref, wut_ref, wub_ref, bu_ref, wn_ref)
    o_ref[...] = jnp.dot(hq2, wo_ref[...], precision=_HIGH,
                         preferred_element_type=_F32) + bo_ref[...]


def _qlast(h_q, agg, na, wut, wub, bu, wn, W_out, b_out):
    grid = (_BN // _RB,)
    return pl.pallas_call(
        _qlast_body,
        grid=grid,
        in_specs=[
            pl.BlockSpec((_RB, _H), lambda i: (i, 0)),
            pl.BlockSpec((_RB, _H), lambda i: (i, 0)),
            pl.BlockSpec((_RB, _SH), lambda i: (i, 0)),
            pl.BlockSpec((_H, _H), lambda i: (0, 0)),
            pl.BlockSpec((_H, _H), lambda i: (0, 0)),
            pl.BlockSpec((1, _H), lambda i: (0, 0)),
            pl.BlockSpec((_SH, _H), lambda i: (0, 0)),
            pl.BlockSpec((_H, _NA * 3), lambda i: (0, 0)),
            pl.BlockSpec((1, _NA * 3), lambda i: (0, 0)),
        ],
        out_specs=[pl.BlockSpec((_RB, _NA * 3), lambda i: (i, 0))],
        out_shape=[jax.ShapeDtypeStruct((_BN, _NA * 3), _F32)],
    )(h_q, agg, na, wut, wub, bu, wn, W_out, b_out)[0]


def kernel(query_points, codes, W_embed, b_embed, Wm, bm, We, Wu, bu, Wn,
           W_out, b_out):
    qflat = query_points.reshape(_BN, 3)
    codes_flat = codes.reshape(_BNG, _CODE)

    idxg, ea_flat, na = _knn(qflat)
    idx_flat = idxg.reshape(_E)
    neg = jnp.full((1, _H), -1e4, _F32)

    h_g, h_q, c_q = _embed(codes_flat, W_embed, b_embed.reshape(1, _H),
                           Wm[0, _H:], bm[0].reshape(1, _H))

    tables = []
    for l in range(_L):
        gp, h_g = _grid_step(h_g, Wm[l, :_H], Wu[l, :_H],
                             bu[l].reshape(1, _H), Wn[l])
        tables.append(gp)

    out = None
    for l in range(_L):
        gathered = _sc_gather(tables[l], idx_flat)
        agg = _edge(gathered, ea_flat, c_q,
                    jnp.concatenate([We[l], neg], axis=0))
        if l + 1 < _L:
            h_q, c_q = _qup(h_q, agg, na, Wu[l, :_H], Wu[l, _H:],
                            bu[l].reshape(1, _H), Wn[l],
                            Wm[l + 1, _H:], bm[l + 1].reshape(1, _H))
        else:
            out = _qlast(h_q, agg, na, Wu[l, :_H], Wu[l, _H:],
                         bu[l].reshape(1, _H), Wn[l],
                         W_out, b_out.reshape(1, _NA * 3))

    return out.reshape(_B, _N, _NA, 3)


# speedup vs baseline: 6.2552x; 1.0034x over previous
import functools

import jax
import jax.numpy as jnp
from jax import lax
from jax.experimental import pallas as pl
from jax.experimental.pallas import tpu as pltpu
from jax.experimental.pallas import tpu_sc as plsc

_B = 2
_N = 4096
_G = 16
_NG = _G ** 3
_CODE = 128
_K = 32
_H = 240
_L = 3
_SH = 9
_NA = 4
_CUTOFF = 8.0
_SPACING = 1.5
_ORIGIN = -(_G - 1) / 2.0 * _SPACING
_BN = _B * _N
_BNG = _B * _NG
_E = _BN * _K

_HP = 256
_QB = 128
_CAND = 128

_OFFS = [
    (-2, -1, 0), (-2, -1, 1), (-2, 0, -1), (-2, 0, 0), (-2, 0, 1),
    (-2, 0, 2), (-2, 1, -1), (-2, 1, 0), (-2, 1, 1), (-2, 1, 2),
    (-2, 2, 0), (-2, 2, 1), (-1, -2, 0), (-1, -2, 1), (-1, -1, -1),
    (-1, -1, 0), (-1, -1, 1), (-1, -1, 2), (-1, 0, -2), (-1, 0, -1),
    (-1, 0, 0), (-1, 0, 1), (-1, 0, 2), (-1, 0, 3), (-1, 1, -2),
    (-1, 1, -1), (-1, 1, 0), (-1, 1, 1), (-1, 1, 2), (-1, 1, 3),
    (-1, 2, -1), (-1, 2, 0), (-1, 2, 1), (-1, 2, 2), (0, -2, 0),
    (0, -2, 1), (0, -1, -1), (0, -1, 0), (0, -1, 1), (0, -1, 2),
    (0, 0, -2), (0, 0, -1), (0, 0, 0), (0, 0, 1), (0, 0, 2), (0, 0, 3),
    (0, 1, -2), (0, 1, -1), (0, 1, 0), (0, 1, 1), (0, 1, 2), (0, 1, 3),
    (0, 2, -1), (0, 2, 0), (0, 2, 1), (0, 2, 2), (0, 3, 0), (0, 3, 1),
    (1, -2, 0), (1, -2, 1), (1, -1, -1), (1, -1, 0), (1, -1, 1),
    (1, -1, 2), (1, 0, -2), (1, 0, -1), (1, 0, 0), (1, 0, 1), (1, 0, 2),
    (1, 0, 3), (1, 1, -2), (1, 1, -1), (1, 1, 0), (1, 1, 1), (1, 1, 2),
    (1, 1, 3), (1, 2, -1), (1, 2, 0), (1, 2, 1), (1, 2, 2), (1, 3, 0),
    (1, 3, 1), (2, -1, -1), (2, -1, 0), (2, -1, 1), (2, -1, 2),
    (2, 0, -1), (2, 0, 0), (2, 0, 1), (2, 0, 2), (2, 1, -1), (2, 1, 0),
    (2, 1, 1), (2, 1, 2), (2, 2, -1), (2, 2, 0), (2, 2, 1), (2, 2, 2),
    (3, 0, 0), (3, 0, 1), (3, 1, 0), (3, 1, 1),
]
_NOFF = len(_OFFS)
_RB = 512
_QB2 = 256
_EB = _QB2 * _K

_NC = 2
_NS = 16
_NW = _NC * _NS
_CH = 128

_F32 = jnp.float32
_HIGH = lax.Precision.HIGHEST


def _silu(x):
    return x * jax.nn.sigmoid(x)


def _rmsnorm(h):
    return h / jnp.sqrt(jnp.mean(h * h, axis=-1, keepdims=True) + 1e-6)


def _knn_body(q_ref, offs_ref, idx_ref, ea_ref, na_ref):
    qx = q_ref[:, 0:1]
    qy = q_ref[:, 1:2]
    qz = q_ref[:, 2:3]

    j = lax.broadcasted_iota(jnp.int32, (_QB, _CAND), 1)
    ox = offs_ref[0:1, :]
    oy = offs_ref[1:2, :]
    oz = offs_ref[2:3, :]

    inv_sp = 1.0 / _SPACING
    fx = jnp.floor((qx - _ORIGIN) * inv_sp)
    fy = jnp.floor((qy - _ORIGIN) * inv_sp)
    fz = jnp.floor((qz - _ORIGIN) * inv_sp)

    cx = jnp.clip(fx + ox, 0.0, _G - 1.0)
    cy = jnp.clip(fy + oy, 0.0, _G - 1.0)
    cz = jnp.clip(fz + oz, 0.0, _G - 1.0)
    rx = (cx * _SPACING + _ORIGIN) - qx
    ry = (cy * _SPACING + _ORIGIN) - qy
    rz = (cz * _SPACING + _ORIGIN) - qz
    d2 = rx * rx + ry * ry + rz * rz

    valid = j < _NOFF
    gidx = (cx * 256.0 + cy * 16.0 + cz).astype(jnp.int32)
    gidx = jnp.where(valid, gidx, 100000 + j)
    big = jnp.float32(3.0e38)
    work = jnp.where(valid, d2, big)

    kiota = lax.broadcasted_iota(jnp.int32, (_QB, _K), 1)
    selg = jnp.zeros((_QB, _K), jnp.int32)
    imax = jnp.int32(2 ** 31 - 1)
    for k in range(_K):
        m = jnp.min(work, axis=1, keepdims=True)
        sg = jnp.min(jnp.where(work == m, gidx, imax), axis=1, keepdims=True)
        work = jnp.where(gidx == sg, big, work)
        selg = jnp.where(kiota == k, sg, selg)

    scx = lax.shift_right_logical(selg, 8).astype(_F32)
    scy = jnp.bitwise_and(lax.shift_right_logical(selg, 4), 15).astype(_F32)
    scz = jnp.bitwise_and(selg, 15).astype(_F32)
    srx = (scx * _SPACING + _ORIGIN) - qx
    sry = (scy * _SPACING + _ORIGIN) - qy
    srz = (scz * _SPACING + _ORIGIN) - qz
    sd2 = srx * srx + sry * sry + srz * srz

    dist = jnp.sqrt(sd2 + 1e-12)
    maskv = (dist <= _CUTOFF).astype(_F32)
    x = srx / dist
    y = sry / dist
    z = srz / dist
    c0 = 0.28209479177387814
    c1 = 0.4886025119029199
    c2 = 1.0925484305920792
    c20 = 0.31539156525252005
    c22 = 0.5462742152960396
    comps = [
        jnp.full((_QB, _K), c0, _F32),
        c1 * y, c1 * z, c1 * x,
        c2 * x * y, c2 * y * z, c20 * (3.0 * z * z - 1.0), c2 * x * z,
        c22 * (x * x - y * y)
    ]

    boff = (pl.program_id(0) * _QB // _N) * _NG
    idx_ref[...] = selg + boff
    ea_ref[...] = jnp.swapaxes(
        jnp.stack(comps + [1.0 - maskv], axis=1), 1, 2).reshape(
            _QB * _K, _SH + 1)
    den = jnp.maximum(jnp.sum(maskv, axis=1, keepdims=True), 1.0)
    na_cols = [jnp.ones((_QB, 1), _F32)]
    for i in range(1, _SH):
        na_cols.append(jnp.sum(comps[i] * maskv, axis=1, keepdims=True) / den)
    na_ref[...] = jnp.concatenate(na_cols, axis=1)


def _knn(qflat):
    pad = (0,) * (_CAND - _NOFF)
    offs = jnp.asarray(
        [[float(o[d]) for o in _OFFS] + list(pad) for d in range(3)], _F32)
    grid = (_BN // _QB,)
    return pl.pallas_call(
        _knn_body,
        grid=grid,
        in_specs=[pl.BlockSpec((_QB, 3), lambda i: (i, 0)),
                  pl.BlockSpec((3, _CAND), lambda i: (0, 0))],
        out_specs=[
            pl.BlockSpec((_QB, _K), lambda i: (i, 0)),
            pl.BlockSpec((_QB * _K, _SH + 1), lambda i: (i, 0)),
            pl.BlockSpec((_QB, _SH), lambda i: (i, 0)),
        ],
        out_shape=[
            jax.ShapeDtypeStruct((_BN, _K), jnp.int32),
            jax.ShapeDtypeStruct((_E, _SH + 1), _F32),
            jax.ShapeDtypeStruct((_BN, _SH), _F32),
        ],
    )(qflat, offs)


def _embed_body(codes_ref, w_ref, b_ref, wmb_ref, bm_ref, hg_ref, hq_ref,
                cq_ref):
    z = jnp.dot(codes_ref[...], w_ref[...], precision=_HIGH,
                preferred_element_type=_F32) + b_ref[...]
    hg_ref[...] = _silu(z)
    hq_row = _silu(b_ref[...])
    hq_ref[...] = jnp.broadcast_to(hq_row, (_RB, _H))
    cq_row = jnp.dot(hq_row, wmb_ref[...], precision=_HIGH,
                     preferred_element_type=_F32) + bm_ref[...]
    cq_ref[...] = jnp.broadcast_to(cq_row, (_RB, _H))


def _embed(codes_flat, W_embed, b_embed, wmb0, bm0):
    grid = (_BNG // _RB,)
    return pl.pallas_call(
        _embed_body,
        grid=grid,
        in_specs=[
            pl.BlockSpec((_RB, _CODE), lambda i: (i, 0)),
            pl.BlockSpec((_CODE, _H), lambda i: (0, 0)),
            pl.BlockSpec((1, _H), lambda i: (0, 0)),
            pl.BlockSpec((_H, _H), lambda i: (0, 0)),
            pl.BlockSpec((1, _H), lambda i: (0, 0)),
        ],
        out_specs=[
            pl.BlockSpec((_RB, _H), lambda i: (i, 0)),
            pl.BlockSpec((_RB, _H), lambda i: (i, 0)),
            pl.BlockSpec((_RB, _H), lambda i: (i, 0)),
        ],
        out_shape=[
            jax.ShapeDtypeStruct((_BNG, _H), _F32),
            jax.ShapeDtypeStruct((_BN, _H), _F32),
            jax.ShapeDtypeStruct((_BN, _H), _F32),
        ],
    )(codes_flat, W_embed, b_embed, wmb0, bm0)


def _grid_body(hg_ref, wmt_ref, wut_ref, bu_ref, wn_ref, gp_ref, hgo_ref):
    hg = hg_ref[...]
    gp = jnp.dot(hg, wmt_ref[...], precision=_HIGH,
                 preferred_element_type=_F32)
    gp_ref[...] = jnp.concatenate(
        [gp, jnp.zeros((_RB, _HP - _H), _F32)], axis=1)
    ug = _silu(jnp.dot(hg, wut_ref[...], precision=_HIGH,
                       preferred_element_type=_F32) + bu_ref[...])
    gg = jax.nn.sigmoid(wn_ref[0:1, :])
    hgo_ref[...] = _rmsnorm(hg + ug * gg)


def _grid_step(h_g, wmt, wut, bu, wn):
    grid = (_BNG // _RB,)
    return pl.pallas_call(
        _grid_body,
        grid=grid,
        in_specs=[
            pl.BlockSpec((_RB, _H), lambda i: (i, 0)),
            pl.BlockSpec((_H, _H), lambda i: (0, 0)),
            pl.BlockSpec((_H, _H), lambda i: (0, 0)),
            pl.BlockSpec((1, _H), lambda i: (0, 0)),
            pl.BlockSpec((_SH, _H), lambda i: (0, 0)),
        ],
        out_specs=[
            pl.BlockSpec((_RB, _HP), lambda i: (i, 0)),
            pl.BlockSpec((_RB, _H), lambda i: (i, 0)),
        ],
        out_shape=[
            jax.ShapeDtypeStruct((_BNG, _HP), _F32),
            jax.ShapeDtypeStruct((_BNG, _H), _F32),
        ],
    )(h_g, wmt, wut, bu, wn)


_GRP = 8


def _sc_gather_body(table_ref, idx_ref, out_ref, idx_v, rows0, rows1, sem0,
                    sem1):
    wid = lax.axis_index("s") * _NC + lax.axis_index("c")
    per_w = _E // _NW
    base = wid * per_w
    pltpu.sync_copy(idx_ref.at[pl.ds(base, per_w)], idx_v)
    bufs = (rows0, rows1)
    sems = (sem0, sem1)

    def group(g, carry):
        cps = []
        for jj in range(_GRP):
            i = g * _GRP + jj
            cp = pltpu.async_copy(
                table_ref.at[idx_v.at[pl.ds(i * _CH, _CH)]],
                bufs[jj % 2], sems[jj % 2])
            cps.append(cp)
            if jj >= 1:
                cps[jj - 1].wait()
                pltpu.sync_copy(
                    bufs[(jj - 1) % 2],
                    out_ref.at[pl.ds(base + (i - 1) * _CH, _CH)])
        cps[_GRP - 1].wait()
        pltpu.sync_copy(
            bufs[(_GRP - 1) % 2],
            out_ref.at[pl.ds(base + (g * _GRP + _GRP - 1) * _CH, _CH)])
        return carry

    lax.fori_loop(0, per_w // _CH // _GRP, group, 0)


@functools.lru_cache(maxsize=1)
def _sc_gather_fn():
    return pl.kernel(
        _sc_gather_body,
        out_type=jax.ShapeDtypeStruct((_E, _HP), _F32),
        mesh=plsc.VectorSubcoreMesh(core_axis_name="c", subcore_axis_name="s"),
        scratch_types=[
            pltpu.VMEM((_E // _NW,), jnp.int32),
            pltpu.VMEM((_CH, _HP), _F32),
            pltpu.VMEM((_CH, _HP), _F32),
            pltpu.SemaphoreType.DMA,
            pltpu.SemaphoreType.DMA,
        ],
    )


def _sc_gather(table, idx):
    return _sc_gather_fn()(table, idx)


def _edge_body(g_ref, ea_ref, cq_ref, we_ref, agg_ref):
    g3 = g_ref[:, :_H].reshape(_QB2, _K, _H)
    pre = g3 + cq_ref[...].reshape(_QB2, 1, _H)
    m = _silu(pre)
    gate = jax.nn.sigmoid(
        jnp.dot(ea_ref[...], we_ref[...], precision=_HIGH,
                preferred_element_type=_F32))
    m = m * gate.reshape(_QB2, _K, _H)
    agg_ref[...] = jnp.sum(m, axis=1)


def _edge(gathered, ea_flat, c_q, we_aug):
    grid = (_BN // _QB2,)
    return pl.pallas_call(
        _edge_body,
        grid=grid,
        in_specs=[
            pl.BlockSpec((_EB, _HP), lambda i: (i, 0)),
            pl.BlockSpec((_EB, _SH + 1), lambda i: (i, 0)),
            pl.BlockSpec((_QB2, _H), lambda i: (i, 0)),
            pl.BlockSpec((_SH + 1, _H), lambda i: (0, 0)),
        ],
        out_specs=[pl.BlockSpec((_QB2, _H), lambda i: (i, 0))],
        out_shape=[jax.ShapeDtypeStruct((_BN, _H), _F32)],
    )(gathered, ea_flat, c_q, we_aug)[0]


def _qup_core(hq_ref, agg_ref, na_ref, wut_ref, wub_ref, bu_ref, wn_ref):
    hq = hq_ref[...]
    uq = _silu(jnp.dot(hq, wut_ref[...], precision=_HIGH,
                       preferred_element_type=_F32)
               + jnp.dot(agg_ref[...], wub_ref[...], precision=_HIGH,
                         preferred_element_type=_F32) + bu_ref[...])
    gq = jax.nn.sigmoid(jnp.dot(na_ref[...], wn_ref[...], precision=_HIGH,
                                preferred_element_type=_F32))
    return _rmsnorm(hq + uq * gq)


def _qup_body(hq_ref, agg_ref, na_ref, wut_ref, wub_ref, bu_ref, wn_ref,
              wmb_ref, bm_ref, hqo_ref, cqo_ref):
    hq2 = _qup_core(hq_ref, agg_ref, na_ref, wut_ref, wub_ref, bu_ref, wn_ref)
    hqo_ref[...] = hq2
    cqo_ref[...] = jnp.dot(hq2, wmb_ref[...], precision=_HIGH,
                           preferred_element_type=_F32) + bm_ref[...]


def _qup(h_q, agg, na, wut, wub, bu, wn, wmb_next, bm_next):
    grid = (_BN // _RB,)
    return pl.pallas_call(
        _qup_body,
        grid=grid,
        in_specs=[
            pl.BlockSpec((_RB, _H), lambda i: (i, 0)),
            pl.BlockSpec((_RB, _H), lambda i: (i, 0)),
            pl.BlockSpec((_RB, _SH), lambda i: (i, 0)),
            pl.BlockSpec((_H, _H), lambda i: (0, 0)),
            pl.BlockSpec((_H, _H), lambda i: (0, 0)),
            pl.BlockSpec((1, _H), lambda i: (0, 0)),
            pl.BlockSpec((_SH, _H), lambda i: (0, 0)),
            pl.BlockSpec((_H, _H), lambda i: (0, 0)),
            pl.BlockSpec((1, _H), lambda i: (0, 0)),
        ],
        out_specs=[
            pl.BlockSpec((_RB, _H), lambda i: (i, 0)),
            pl.BlockSpec((_RB, _H), lambda i: (i, 0)),
        ],
        out_shape=[
            jax.ShapeDtypeStruct((_BN, _H), _F32),
            jax.ShapeDtypeStruct((_BN, _H), _F32),
        ],
    )(h_q, agg, na, wut, wub, bu, wn, wmb_next, bm_next)


def _qlast_body(hq_ref, agg_ref, na_ref, wut_ref, wub_ref, bu_ref, wn_ref,
                wo_ref, bo_ref, o_ref):
    hq2 = _qup_core(hq_ref, agg_ref, na_ref, wut_ref, wub_ref, bu_ref, wn_ref)
    o_ref[...] = jnp.dot(hq2, wo_ref[...], precision=_HIGH,
                         preferred_element_type=_F32) + bo_ref[...]


def _qlast(h_q, agg, na, wut, wub, bu, wn, W_out, b_out):
    grid = (_BN // _RB,)
    return pl.pallas_call(
        _qlast_body,
        grid=grid,
        in_specs=[
            pl.BlockSpec((_RB, _H), lambda i: (i, 0)),
            pl.BlockSpec((_RB, _H), lambda i: (i, 0)),
            pl.BlockSpec((_RB, _SH), lambda i: (i, 0)),
            pl.BlockSpec((_H, _H), lambda i: (0, 0)),
            pl.BlockSpec((_H, _H), lambda i: (0, 0)),
            pl.BlockSpec((1, _H), lambda i: (0, 0)),
            pl.BlockSpec((_SH, _H), lambda i: (0, 0)),
            pl.BlockSpec((_H, _NA * 3), lambda i: (0, 0)),
            pl.BlockSpec((1, _NA * 3), lambda i: (0, 0)),
        ],
        out_specs=[pl.BlockSpec((_RB, _NA * 3), lambda i: (i, 0))],
        out_shape=[jax.ShapeDtypeStruct((_BN, _NA * 3), _F32)],
    )(h_q, agg, na, wut, wub, bu, wn, W_out, b_out)[0]


def kernel(query_points, codes, W_embed, b_embed, Wm, bm, We, Wu, bu, Wn,
           W_out, b_out):
    qflat = query_points.reshape(_BN, 3)
    codes_flat = codes.reshape(_BNG, _CODE)

    idxg, ea_flat, na = _knn(qflat)
    idx_flat = idxg.reshape(_E)
    neg = jnp.full((1, _H), -1e4, _F32)

    h_g, h_q, c_q = _embed(codes_flat, W_embed, b_embed.reshape(1, _H),
                           Wm[0, _H:], bm[0].reshape(1, _H))

    tables = []
    for l in range(_L):
        gp, h_g = _grid_step(h_g, Wm[l, :_H], Wu[l, :_H],
                             bu[l].reshape(1, _H), Wn[l])
        tables.append(gp)

    out = None
    for l in range(_L):
        gathered = _sc_gather(tables[l], idx_flat)
        agg = _edge(gathered, ea_flat, c_q,
                    jnp.concatenate([We[l], neg], axis=0))
        if l + 1 < _L:
            h_q, c_q = _qup(h_q, agg, na, Wu[l, :_H], Wu[l, _H:],
                            bu[l].reshape(1, _H), Wn[l],
                            Wm[l + 1, _H:], bm[l + 1].reshape(1, _H))
        else:
            out = _qlast(h_q, agg, na, Wu[l, :_H], Wu[l, _H:],
                         bu[l].reshape(1, _H), Wn[l],
                         W_out, b_out.reshape(1, _NA * 3))

    return out.reshape(_B, _N, _NA, 3)
